# Initial kernel scaffold; baseline (speedup 1.0000x reference)
import jax
import jax.numpy as jnp
from jax.experimental import pallas as pl


def kernel(reid_t0, boxes_t0, scores_t0, reid_t1, boxes_t1, scores_t1, edge_index_t0, edge_index_t1, assc_edge_index, id_t0, id_t1, W_e, W_m, W_n, W_a):
    raise NotImplementedError("write your pallas kernel here")


# speedup vs baseline: 6.0353x; 6.0353x over previous
import jax
import jax.numpy as jnp
from jax import lax
from jax.experimental import pallas as pl
from jax.experimental.pallas import tpu as pltpu
from jax.experimental.pallas import tpu_sc as plsc

_N = 10000
_E = 160000
_EA = 160000
_D = 128
_DE = 64

_NC = 2
_NS = 16
_NW = _NC * _NS
_CHUNK = 128
_NCHUNK_E = _E // _CHUNK
_NCHUNK_A = _EA // _CHUNK
_ROWBLK = 128
_NB = (_N + _ROWBLK - 1) // _ROWBLK
_NPAD = 10240
_NROWZ = _NPAD // _NS


def _sc_mesh():
    return plsc.VectorSubcoreMesh(core_axis_name="c", subcore_axis_name="s",
                                  num_cores=_NC, num_subcores=_NS)


_SC_PARAMS = pltpu.CompilerParams(needs_layout_passes=False)


def _xin_y_body(reid_ref, s_ref, wm1_ref, xin_ref, y_ref):
    x = reid_ref[...] * s_ref[...]
    xin_ref[...] = x
    y_ref[...] = jnp.dot(x, wm1_ref[...], preferred_element_type=jnp.float32)


def _stage_a(reid, scores, wm1):
    return pl.pallas_call(
        _xin_y_body,
        grid=(_NB,),
        in_specs=[pl.BlockSpec((_ROWBLK, _D), lambda i: (i, 0)),
                  pl.BlockSpec((_ROWBLK, 1), lambda i: (i, 0)),
                  pl.BlockSpec((_D, _D), lambda i: (0, 0))],
        out_specs=[pl.BlockSpec((_ROWBLK, _D), lambda i: (i, 0)),
                   pl.BlockSpec((_ROWBLK, _D), lambda i: (i, 0))],
        out_shape=[jax.ShapeDtypeStruct((_N, _D), jnp.float32),
                   jax.ShapeDtypeStruct((_N, _D), jnp.float32)],
    )(reid, scores.reshape(_N, 1), wm1)


def _geom_body(boxes0, boxes1, s0, d0, s1, d1, geom_hbm,
               table_v, src_v, dst_v, gbuf_v):
    cid = lax.axis_index("c")
    sid = lax.axis_index("s")
    wid = sid * _NC + cid
    for f in range(2):
        bx = boxes0 if f == 0 else boxes1
        sh = s0 if f == 0 else s1
        dh = d0 if f == 0 else d1
        pltpu.sync_copy(bx, table_v)

        def chunk_body(i, _, f=f, sh=sh, dh=dh):
            k = wid + i * _NW
            base = k * _CHUNK
            pltpu.sync_copy(sh.at[pl.ds(base, _CHUNK)], src_v)
            pltpu.sync_copy(dh.at[pl.ds(base, _CHUNK)], dst_v)

            def vec_body(j, _):
                sl = pl.ds(j * 16, 16)
                s16 = src_v[sl] * 4
                d16 = dst_v[sl] * 4
                for comp in range(4):
                    a = plsc.load_gather(table_v, [s16 + comp])
                    b = plsc.load_gather(table_v, [d16 + comp])
                    gbuf_v[comp, sl] = a - b
                return 0

            lax.fori_loop(0, _CHUNK // 16, vec_body, 0)
            pltpu.sync_copy(gbuf_v, geom_hbm.at[f, :, pl.ds(base, _CHUNK)])
            return 0

        nk = (_NCHUNK_E - wid + _NW - 1) // _NW
        lax.fori_loop(0, nk, chunk_body, 0)


def _stage_b(boxes0, boxes1, s0, d0, s1, d1):
    return pl.kernel(
        _geom_body,
        out_type=jax.ShapeDtypeStruct((2, 4, _E), jnp.float32),
        mesh=_sc_mesh(),
        compiler_params=_SC_PARAMS,
        scratch_types=[pltpu.VMEM((_N * 4,), jnp.float32),
                       pltpu.VMEM((_CHUNK,), jnp.int32),
                       pltpu.VMEM((_CHUNK,), jnp.int32),
                       pltpu.VMEM((4, _CHUNK), jnp.float32)],
    )(boxes0.reshape(_N * 4), boxes1.reshape(_N * 4), s0, d0, s1, d1)


_BE = 1280


def _cmat_body(g_ref, we_ref, wm2_ref, c_ref):
    g = g_ref[0]
    e = lax.dot_general(g, we_ref[...], (((0,), (0,)), ((), ())),
                        preferred_element_type=jnp.float32)
    e = jnp.maximum(e, 0.0)
    c_ref[...] = jnp.dot(e, wm2_ref[...], preferred_element_type=jnp.float32)


def _stage_c(geom, f, we, wm2):
    return pl.pallas_call(
        _cmat_body,
        grid=(_E // _BE,),
        in_specs=[pl.BlockSpec((1, 4, _BE), lambda i, f=f: (f, 0, i)),
                  pl.BlockSpec((4, _DE), lambda i: (0, 0)),
                  pl.BlockSpec((_DE, _D), lambda i: (0, 0))],
        out_specs=pl.BlockSpec((_BE, _D), lambda i: (i, 0)),
        out_shape=jax.ShapeDtypeStruct((_E, _D), jnp.float32),
    )(geom, we, wm2)


def _agg_body(y0, y1, c0, c1, s0, d0, s1, d1, aggp_hbm,
              acc_sh, zbuf_v, src_v, dst_v, ybuf_v, cbuf_v, gsem, csem):
    cid = lax.axis_index("c")
    sid = lax.axis_index("s")

    def zrow(r, _):
        for cb in range(8):
            zbuf_v[r, pl.ds(cb * 16, 16)] = jnp.zeros((16,), jnp.float32)
        return 0

    lax.fori_loop(0, _NROWZ // 10, zrow, 0)

    for f in range(2):
        yh = y0 if f == 0 else y1
        ch = c0 if f == 0 else c1
        sh = s0 if f == 0 else s1
        dh = d0 if f == 0 else d1

        for t in range(10):
            pltpu.sync_copy(
                zbuf_v, acc_sh.at[pl.ds(sid * _NROWZ + t * (_NROWZ // 10),
                                        _NROWZ // 10), :])
        plsc.subcore_barrier()

        half = _NCHUNK_E // _NC
        lo = cid * half + sid
        nk = (half - sid + _NS - 1) // _NS

        def chunk_body(i, _, yh=yh, ch=ch, sh=sh, dh=dh, lo=lo):
            k = lo + i * _NS
            base = k * _CHUNK
            pltpu.sync_copy(sh.at[pl.ds(base, _CHUNK)], src_v)
            pltpu.sync_copy(dh.at[pl.ds(base, _CHUNK)], dst_v)
            gcp = pltpu.async_copy(yh.at[src_v], ybuf_v, gsem)
            ccp = pltpu.async_copy(ch.at[pl.ds(base, _CHUNK), :], cbuf_v, csem)
            gcp.wait()
            ccp.wait()

            def row_body(r, _):
                for cb in range(8):
                    sl = pl.ds(cb * 16, 16)
                    cbuf_v[r, sl] = jnp.maximum(ybuf_v[r, sl] + cbuf_v[r, sl],
                                                0.0)
                return 0

            lax.fori_loop(0, _CHUNK, row_body, 0)
            pltpu.sync_copy(cbuf_v, acc_sh.at[dst_v], add=True)
            return 0

        lax.fori_loop(0, nk, chunk_body, 0)
        plsc.subcore_barrier()
        pltpu.sync_copy(acc_sh.at[pl.ds(sid * _NROWZ, _NROWZ), :],
                        aggp_hbm.at[f, cid, pl.ds(sid * _NROWZ, _NROWZ), :])
        plsc.subcore_barrier()


def _stage_d(y0, y1, c0, c1, s0, d0, s1, d1):
    return pl.kernel(
        _agg_body,
        out_type=jax.ShapeDtypeStruct((2, _NC, _NPAD, _D), jnp.float32),
        mesh=_sc_mesh(),
        compiler_params=_SC_PARAMS,
        scratch_types=[pltpu.VMEM_SHARED((_NPAD, _D), jnp.float32),
                       pltpu.VMEM((_NROWZ // 10, _D), jnp.float32),
                       pltpu.VMEM((_CHUNK,), jnp.int32),
                       pltpu.VMEM((_CHUNK,), jnp.int32),
                       pltpu.VMEM((_CHUNK, _D), jnp.float32),
                       pltpu.VMEM((_CHUNK, _D), jnp.float32),
                       pltpu.SemaphoreType.DMA,
                       pltpu.SemaphoreType.DMA],
    )(y0, y1, c0, c1, s0, d0, s1, d1)


def _h_body(xin_ref, a0_ref, a1_ref, wn1_ref, wn2_ref, wa_ref, p_ref):
    x = xin_ref[...]
    agg = a0_ref[0, 0] + a1_ref[0, 0]
    pre = (jnp.dot(x, wn1_ref[...], preferred_element_type=jnp.float32)
           + jnp.dot(agg, wn2_ref[...], preferred_element_type=jnp.float32))
    h = jnp.maximum(pre, 0.0)
    nrm = jnp.sqrt(jnp.sum(h * h, axis=1, keepdims=True)) + 1e-8
    hn = h / nrm
    p_ref[...] = jnp.dot(hn, wa_ref[...], preferred_element_type=jnp.float32)


def _stage_e(xin, aggp, f, wn1, wn2, wa):
    return pl.pallas_call(
        _h_body,
        grid=(_NB,),
        in_specs=[pl.BlockSpec((_ROWBLK, _D), lambda i: (i, 0)),
                  pl.BlockSpec((1, 1, _ROWBLK, _D), lambda i, f=f: (f, 0, i, 0)),
                  pl.BlockSpec((1, 1, _ROWBLK, _D), lambda i, f=f: (f, 1, i, 0)),
                  pl.BlockSpec((_D, _D), lambda i: (0, 0)),
                  pl.BlockSpec((_D, _D), lambda i: (0, 0)),
                  pl.BlockSpec((_D, 1), lambda i: (0, 0))],
        out_specs=pl.BlockSpec((_ROWBLK, 1), lambda i: (i, 0)),
        out_shape=jax.ShapeDtypeStruct((_N, 1), jnp.float32),
    )(xin, aggp, aggp, wn1, wn2, wa)


def _assoc_body(p0, p1, id0, id1, sa, da, lg_hbm, lab_hbm,
                p0t, p1t, id0t, id1t, s_v, d_v, lbuf, labbuf):
    cid = lax.axis_index("c")
    sid = lax.axis_index("s")
    wid = sid * _NC + cid
    pltpu.sync_copy(p0, p0t)
    pltpu.sync_copy(p1, p1t)
    pltpu.sync_copy(id0, id0t)
    pltpu.sync_copy(id1, id1t)

    def chunk_body(i, _):
        k = wid + i * _NW
        base = k * _CHUNK
        pltpu.sync_copy(sa.at[pl.ds(base, _CHUNK)], s_v)
        pltpu.sync_copy(da.at[pl.ds(base, _CHUNK)], d_v)

        def vec_body(j, _):
            sl = pl.ds(j * 16, 16)
            s16 = s_v[sl]
            d16 = d_v[sl]
            pa = plsc.load_gather(p0t, [s16])
            pb = plsc.load_gather(p1t, [d16])
            lbuf[sl] = pa + pb
            ia = plsc.load_gather(id0t, [s16])
            ib = plsc.load_gather(id1t, [d16])
            labbuf[sl] = jnp.where(ia == ib, 1.0, 0.0)
            return 0

        lax.fori_loop(0, _CHUNK // 16, vec_body, 0)
        pltpu.sync_copy(lbuf, lg_hbm.at[pl.ds(base, _CHUNK)])
        pltpu.sync_copy(labbuf, lab_hbm.at[pl.ds(base, _CHUNK)])
        return 0

    nk = (_NCHUNK_A - wid + _NW - 1) // _NW
    lax.fori_loop(0, nk, chunk_body, 0)


def _stage_f(p0, p1, id0, id1, sa, da):
    return pl.kernel(
        _assoc_body,
        out_type=(jax.ShapeDtypeStruct((_EA,), jnp.float32),
                  jax.ShapeDtypeStruct((_EA,), jnp.float32)),
        mesh=_sc_mesh(),
        compiler_params=_SC_PARAMS,
        scratch_types=[pltpu.VMEM((_N,), jnp.float32),
                       pltpu.VMEM((_N,), jnp.float32),
                       pltpu.VMEM((_N,), jnp.int32),
                       pltpu.VMEM((_N,), jnp.int32),
                       pltpu.VMEM((_CHUNK,), jnp.int32),
                       pltpu.VMEM((_CHUNK,), jnp.int32),
                       pltpu.VMEM((_CHUNK,), jnp.float32),
                       pltpu.VMEM((_CHUNK,), jnp.float32)],
    )(p0, p1, id0, id1, sa, da)


def _bce_body(lg_ref, lab_ref, out_ref):
    l = lg_ref[...]
    lab = lab_ref[...]
    b = (jnp.maximum(l, 0.0) - l * lab
         + jnp.log(1.0 + jnp.exp(-jnp.abs(l))))
    out_ref[...] = jnp.sum(b, axis=(0, 1), keepdims=True) * (1.0 / _EA)


def _stage_g(lg, lab):
    return pl.pallas_call(
        _bce_body,
        grid=(1,),
        in_specs=[pl.BlockSpec((_NCHUNK_A, _CHUNK), lambda i: (0, 0)),
                  pl.BlockSpec((_NCHUNK_A, _CHUNK), lambda i: (0, 0))],
        out_specs=pl.BlockSpec((1, 1), lambda i: (0, 0)),
        out_shape=jax.ShapeDtypeStruct((1, 1), jnp.float32),
    )(lg.reshape(_NCHUNK_A, _CHUNK), lab.reshape(_NCHUNK_A, _CHUNK))


def kernel(reid_t0, boxes_t0, scores_t0, reid_t1, boxes_t1, scores_t1,
           edge_index_t0, edge_index_t1, assc_edge_index, id_t0, id_t1,
           W_e, W_m, W_n, W_a):
    ed0 = edge_index_t0.astype(jnp.int32)
    ed1 = edge_index_t1.astype(jnp.int32)
    assc = assc_edge_index.astype(jnp.int32)
    id0 = id_t0.astype(jnp.int32)
    id1 = id_t1.astype(jnp.int32)

    wm1 = W_m[:_D]
    wm2 = W_m[_D:]
    wn1 = W_n[:_D]
    wn2 = W_n[_D:]
    wa0 = W_a[:_D]
    wa1 = W_a[_D:]

    s0, d0 = ed0[0], ed0[1]
    s1, d1 = ed1[0], ed1[1]
    sa, da = assc[0], assc[1]

    x0, y0 = _stage_a(reid_t0, scores_t0, wm1)
    x1, y1 = _stage_a(reid_t1, scores_t1, wm1)
    geom = _stage_b(boxes_t0, boxes_t1, s0, d0, s1, d1)
    c0 = _stage_c(geom, 0, W_e, wm2)
    c1 = _stage_c(geom, 1, W_e, wm2)
    aggp = _stage_d(y0, y1, c0, c1, s0, d0, s1, d1)
    p0 = _stage_e(x0, aggp, 0, wn1, wn2, wa0)
    p1 = _stage_e(x1, aggp, 1, wn1, wn2, wa1)
    lg, lab = _stage_f(p0.reshape(_N), p1.reshape(_N), id0, id1, sa, da)
    res = _stage_g(lg, lab)
    return res[0, 0]


# speedup vs baseline: 7.4683x; 1.2374x over previous
import jax
import jax.numpy as jnp
from jax import lax
from jax.experimental import pallas as pl
from jax.experimental.pallas import tpu as pltpu
from jax.experimental.pallas import tpu_sc as plsc

_N = 10000
_E = 160000
_EA = 160000
_D = 128
_DE = 64

_NC = 2
_NS = 16
_NW = _NC * _NS
_CHUNK = 128
_NCHUNK_E = _E // _CHUNK
_NCHUNK_A = _EA // _CHUNK
_ROWBLK = 128
_NB = (_N + _ROWBLK - 1) // _ROWBLK
_NPAD = 10240
_NROWZ = _NPAD // _NS


def _sc_mesh():
    return plsc.VectorSubcoreMesh(core_axis_name="c", subcore_axis_name="s",
                                  num_cores=_NC, num_subcores=_NS)


_SC_PARAMS = pltpu.CompilerParams(needs_layout_passes=False)


def _pack_bf16_pairs(a):
    lo = jax.lax.bitcast_convert_type(
        a[:, :_D // 2].astype(jnp.bfloat16), jnp.uint16).astype(jnp.uint32)
    hi = jax.lax.bitcast_convert_type(
        a[:, _D // 2:].astype(jnp.bfloat16), jnp.uint16).astype(jnp.uint32)
    return jax.lax.bitcast_convert_type(lo | (hi << 16), jnp.int32)


def _xin_y_body(reid_ref, s_ref, wm1_ref, xin_ref, y_ref):
    x = reid_ref[...] * s_ref[...]
    xin_ref[...] = x
    y_ref[...] = jnp.dot(x, wm1_ref[...], preferred_element_type=jnp.float32)


def _stage_a(reid, scores, wm1):
    return pl.pallas_call(
        _xin_y_body,
        grid=(_NB,),
        in_specs=[pl.BlockSpec((_ROWBLK, _D), lambda i: (i, 0)),
                  pl.BlockSpec((_ROWBLK, 1), lambda i: (i, 0)),
                  pl.BlockSpec((_D, _D), lambda i: (0, 0))],
        out_specs=[pl.BlockSpec((_ROWBLK, _D), lambda i: (i, 0)),
                   pl.BlockSpec((_ROWBLK, _D), lambda i: (i, 0))],
        out_shape=[jax.ShapeDtypeStruct((_N, _D), jnp.float32),
                   jax.ShapeDtypeStruct((_N, _D), jnp.float32)],
    )(reid, scores.reshape(_N, 1), wm1)


def _geom_body(boxes0, boxes1, s0, d0, s1, d1, geom_hbm,
               table_v, src_v, dst_v, gbuf_v):
    cid = lax.axis_index("c")
    sid = lax.axis_index("s")
    wid = sid * _NC + cid
    for f in range(2):
        bx = boxes0 if f == 0 else boxes1
        sh = s0 if f == 0 else s1
        dh = d0 if f == 0 else d1
        pltpu.sync_copy(bx, table_v)

        def chunk_body(i, _, f=f, sh=sh, dh=dh):
            k = wid + i * _NW
            base = k * _CHUNK
            pltpu.sync_copy(sh.at[pl.ds(base, _CHUNK)], src_v)
            pltpu.sync_copy(dh.at[pl.ds(base, _CHUNK)], dst_v)

            def vec_body(j, _):
                sl = pl.ds(j * 16, 16)
                s16 = src_v[sl] * 4
                d16 = dst_v[sl] * 4
                for comp in range(4):
                    a = plsc.load_gather(table_v, [s16 + comp])
                    b = plsc.load_gather(table_v, [d16 + comp])
                    gbuf_v[comp, sl] = a - b
                return 0

            lax.fori_loop(0, _CHUNK // 16, vec_body, 0)
            pltpu.sync_copy(gbuf_v, geom_hbm.at[f, :, pl.ds(base, _CHUNK)])
            return 0

        nk = (_NCHUNK_E - wid + _NW - 1) // _NW
        lax.fori_loop(0, nk, chunk_body, 0)


def _stage_b(boxes0, boxes1, s0, d0, s1, d1):
    return pl.kernel(
        _geom_body,
        out_type=jax.ShapeDtypeStruct((2, 4, _E), jnp.float32),
        mesh=_sc_mesh(),
        compiler_params=_SC_PARAMS,
        scratch_types=[pltpu.VMEM((_N * 4,), jnp.float32),
                       pltpu.VMEM((_CHUNK,), jnp.int32),
                       pltpu.VMEM((_CHUNK,), jnp.int32),
                       pltpu.VMEM((4, _CHUNK), jnp.float32)],
    )(boxes0.reshape(_N * 4), boxes1.reshape(_N * 4), s0, d0, s1, d1)


_BE = 1280


def _cmat_body(g_ref, we_ref, wm2_ref, c_ref):
    g = g_ref[0]
    e = lax.dot_general(g, we_ref[...], (((0,), (0,)), ((), ())),
                        preferred_element_type=jnp.float32)
    e = jnp.maximum(e, 0.0)
    c = jnp.dot(e, wm2_ref[...], preferred_element_type=jnp.float32)
    c_ref[...] = _pack_bf16_pairs(c)


def _stage_c(geom, f, we, wm2):
    return pl.pallas_call(
        _cmat_body,
        grid=(_E // _BE,),
        in_specs=[pl.BlockSpec((1, 4, _BE), lambda i, f=f: (f, 0, i)),
                  pl.BlockSpec((4, _DE), lambda i: (0, 0)),
                  pl.BlockSpec((_DE, _D), lambda i: (0, 0))],
        out_specs=pl.BlockSpec((_BE, _D // 2), lambda i: (i, 0)),
        out_shape=jax.ShapeDtypeStruct((_E, _D // 2), jnp.int32),
    )(geom, we, wm2)


_DCH = 64
_DNCH = _E // _DCH
_DPAIRS = (_DNCH // _NW + 2) // 2 + 1


def _agg_body(y0, y1, c0, c1, s0, d0, s1, d1, aggp_hbm,
              acc_sh, mb, sb0, sb1, db0, db1, yb0, yb1, cb0, cb1,
              isem0, isem1, gsem0, gsem1, ssem):
    cid = lax.axis_index("c")
    sid = lax.axis_index("s")
    wid = sid * _NC + cid

    slots = ((sb0, db0, yb0, cb0, isem0, gsem0),
             (sb1, db1, yb1, cb1, isem1, gsem1))

    for f in range(2):
        yh = y0 if f == 0 else y1
        ch = c0 if f == 0 else c1
        sh = s0 if f == 0 else s1
        dh = d0 if f == 0 else d1

        n = (_DNCH - wid + _NW - 1) // _NW

        def base_of(j):
            return (wid + j * _NW) * _DCH

        def fire_idx(j, sl):
            sb, db = sl[0], sl[1]

            @pl.when(j < n)
            def _():
                b = base_of(j)
                pltpu.async_copy(sh.at[pl.ds(b, _DCH)], sb, sl[4])
                pltpu.async_copy(dh.at[pl.ds(b, _DCH)], db, sl[4])

        def wait_idx(j, sl):
            @pl.when(j < n)
            def _():
                pltpu.make_async_copy(sh.at[pl.ds(0, _DCH)], sl[0], sl[4]).wait()
                pltpu.make_async_copy(dh.at[pl.ds(0, _DCH)], sl[1], sl[4]).wait()

        def fire_data(j, sl):
            @pl.when(j < n)
            def _():
                pltpu.async_copy(yh.at[sl[0]], sl[2], sl[5])
                pltpu.async_copy(ch.at[pl.ds(base_of(j), _DCH), :], sl[3], sl[5])

        def wait_data(j, sl):
            @pl.when(j < n)
            def _():
                pltpu.make_async_copy(yh.at[sl[0]], sl[2], sl[5]).wait()
                pltpu.make_async_copy(ch.at[pl.ds(0, _DCH), :], sl[3], sl[5]).wait()

        def compute_scatter(j, sl, sl_prev):
            yb, cb = sl[2], sl[3]

            @pl.when(jnp.logical_and(j >= 1, j < n))
            def _():
                pltpu.make_async_copy(mb, acc_sh.at[sl_prev[1]], ssem).wait()

            @pl.when(j < n)
            def _():
                def row_body(r, _):
                    for g in range(4):
                        gs = pl.ds(g * 16, 16)
                        hs = pl.ds(64 + g * 16, 16)
                        vc = cb[r, gs]
                        lo = yb[r, gs] + plsc.bitcast(vc << 16, jnp.float32)
                        hi = yb[r, hs] + plsc.bitcast(vc, jnp.float32)
                        mb[r, gs] = jnp.maximum(lo, 0.0)
                        mb[r, hs] = jnp.maximum(hi, 0.0)
                    return 0

                lax.fori_loop(0, _DCH, row_body, 0)
                pltpu.async_copy(mb, acc_sh.at[sl[1]], ssem, add=True)

        def zrow(r, _):
            for g in range(8):
                mb[r, pl.ds(g * 16, 16)] = jnp.zeros((16,), jnp.float32)
            return 0

        lax.fori_loop(0, _DCH, zrow, 0)
        for t in range(_NROWZ // _DCH):
            pltpu.sync_copy(
                mb, acc_sh.at[pl.ds(sid * _NROWZ + t * _DCH, _DCH), :])
        plsc.subcore_barrier()

        fire_idx(0, slots[0])
        wait_idx(0, slots[0])
        fire_data(0, slots[0])
        fire_idx(1, slots[1])

        def pair_body(i, _):
            for b in range(2):
                j = 2 * i + b
                wait_idx(j + 1, slots[1 - b])
                fire_data(j + 1, slots[1 - b])
                wait_data(j, slots[b])
                compute_scatter(j, slots[b], slots[1 - b])
                fire_idx(j + 2, slots[b])
            return 0

        lax.fori_loop(0, _DPAIRS, pair_body, 0)

        @pl.when(n > 0)
        def _():
            pltpu.make_async_copy(mb, acc_sh.at[db0], ssem).wait()

        plsc.subcore_barrier()
        pltpu.sync_copy(acc_sh.at[pl.ds(sid * _NROWZ, _NROWZ), :],
                        aggp_hbm.at[f, cid, pl.ds(sid * _NROWZ, _NROWZ), :])
        plsc.subcore_barrier()


def _stage_d(y0, y1, c0, c1, s0, d0, s1, d1):
    return pl.kernel(
        _agg_body,
        out_type=jax.ShapeDtypeStruct((2, _NC, _NPAD, _D), jnp.float32),
        mesh=_sc_mesh(),
        compiler_params=_SC_PARAMS,
        scratch_types=[pltpu.VMEM_SHARED((_NPAD, _D), jnp.float32),
                       pltpu.VMEM((_DCH, _D), jnp.float32),
                       pltpu.VMEM((_DCH,), jnp.int32),
                       pltpu.VMEM((_DCH,), jnp.int32),
                       pltpu.VMEM((_DCH,), jnp.int32),
                       pltpu.VMEM((_DCH,), jnp.int32),
                       pltpu.VMEM((_DCH, _D), jnp.float32),
                       pltpu.VMEM((_DCH, _D), jnp.float32),
                       pltpu.VMEM((_DCH, _D // 2), jnp.int32),
                       pltpu.VMEM((_DCH, _D // 2), jnp.int32),
                       pltpu.SemaphoreType.DMA,
                       pltpu.SemaphoreType.DMA,
                       pltpu.SemaphoreType.DMA,
                       pltpu.SemaphoreType.DMA,
                       pltpu.SemaphoreType.DMA],
    )(y0, y1, c0, c1, s0, d0, s1, d1)


def _h_body(xin_ref, a0_ref, a1_ref, wn1_ref, wn2_ref, wa_ref, p_ref):
    x = xin_ref[...]
    agg = a0_ref[0, 0] + a1_ref[0, 0]
    pre = (jnp.dot(x, wn1_ref[...], preferred_element_type=jnp.float32)
           + jnp.dot(agg, wn2_ref[...], preferred_element_type=jnp.float32))
    h = jnp.maximum(pre, 0.0)
    nrm = jnp.sqrt(jnp.sum(h * h, axis=1, keepdims=True)) + 1e-8
    hn = h / nrm
    p_ref[...] = jnp.dot(hn, wa_ref[...], preferred_element_type=jnp.float32)


def _stage_e(xin, aggp, f, wn1, wn2, wa):
    return pl.pallas_call(
        _h_body,
        grid=(_NB,),
        in_specs=[pl.BlockSpec((_ROWBLK, _D), lambda i: (i, 0)),
                  pl.BlockSpec((1, 1, _ROWBLK, _D), lambda i, f=f: (f, 0, i, 0)),
                  pl.BlockSpec((1, 1, _ROWBLK, _D), lambda i, f=f: (f, 1, i, 0)),
                  pl.BlockSpec((_D, _D), lambda i: (0, 0)),
                  pl.BlockSpec((_D, _D), lambda i: (0, 0)),
                  pl.BlockSpec((_D, 1), lambda i: (0, 0))],
        out_specs=pl.BlockSpec((_ROWBLK, 1), lambda i: (i, 0)),
        out_shape=jax.ShapeDtypeStruct((_N, 1), jnp.float32),
    )(xin, aggp, aggp, wn1, wn2, wa)


def _assoc_body(p0, p1, id0, id1, sa, da, lg_hbm, lab_hbm,
                p0t, p1t, id0t, id1t, s_v, d_v, lbuf, labbuf):
    cid = lax.axis_index("c")
    sid = lax.axis_index("s")
    wid = sid * _NC + cid
    pltpu.sync_copy(p0, p0t)
    pltpu.sync_copy(p1, p1t)
    pltpu.sync_copy(id0, id0t)
    pltpu.sync_copy(id1, id1t)

    def chunk_body(i, _):
        k = wid + i * _NW
        base = k * _CHUNK
        pltpu.sync_copy(sa.at[pl.ds(base, _CHUNK)], s_v)
        pltpu.sync_copy(da.at[pl.ds(base, _CHUNK)], d_v)

        def vec_body(j, _):
            sl = pl.ds(j * 16, 16)
            s16 = s_v[sl]
            d16 = d_v[sl]
            pa = plsc.load_gather(p0t, [s16])
            pb = plsc.load_gather(p1t, [d16])
            lbuf[sl] = pa + pb
            ia = plsc.load_gather(id0t, [s16])
            ib = plsc.load_gather(id1t, [d16])
            labbuf[sl] = jnp.where(ia == ib, 1.0, 0.0)
            return 0

        lax.fori_loop(0, _CHUNK // 16, vec_body, 0)
        pltpu.sync_copy(lbuf, lg_hbm.at[pl.ds(base, _CHUNK)])
        pltpu.sync_copy(labbuf, lab_hbm.at[pl.ds(base, _CHUNK)])
        return 0

    nk = (_NCHUNK_A - wid + _NW - 1) // _NW
    lax.fori_loop(0, nk, chunk_body, 0)


def _stage_f(p0, p1, id0, id1, sa, da):
    return pl.kernel(
        _assoc_body,
        out_type=(jax.ShapeDtypeStruct((_EA,), jnp.float32),
                  jax.ShapeDtypeStruct((_EA,), jnp.float32)),
        mesh=_sc_mesh(),
        compiler_params=_SC_PARAMS,
        scratch_types=[pltpu.VMEM((_N,), jnp.float32),
                       pltpu.VMEM((_N,), jnp.float32),
                       pltpu.VMEM((_N,), jnp.int32),
                       pltpu.VMEM((_N,), jnp.int32),
                       pltpu.VMEM((_CHUNK,), jnp.int32),
                       pltpu.VMEM((_CHUNK,), jnp.int32),
                       pltpu.VMEM((_CHUNK,), jnp.float32),
                       pltpu.VMEM((_CHUNK,), jnp.float32)],
    )(p0, p1, id0, id1, sa, da)


def _bce_body(lg_ref, lab_ref, out_ref):
    l = lg_ref[...]
    lab = lab_ref[...]
    b = (jnp.maximum(l, 0.0) - l * lab
         + jnp.log(1.0 + jnp.exp(-jnp.abs(l))))
    out_ref[...] = jnp.sum(b, axis=(0, 1), keepdims=True) * (1.0 / _EA)


def _stage_g(lg, lab):
    return pl.pallas_call(
        _bce_body,
        grid=(1,),
        in_specs=[pl.BlockSpec((_NCHUNK_A, _CHUNK), lambda i: (0, 0)),
                  pl.BlockSpec((_NCHUNK_A, _CHUNK), lambda i: (0, 0))],
        out_specs=pl.BlockSpec((1, 1), lambda i: (0, 0)),
        out_shape=jax.ShapeDtypeStruct((1, 1), jnp.float32),
    )(lg.reshape(_NCHUNK_A, _CHUNK), lab.reshape(_NCHUNK_A, _CHUNK))


def kernel(reid_t0, boxes_t0, scores_t0, reid_t1, boxes_t1, scores_t1,
           edge_index_t0, edge_index_t1, assc_edge_index, id_t0, id_t1,
           W_e, W_m, W_n, W_a):
    ed0 = edge_index_t0.astype(jnp.int32)
    ed1 = edge_index_t1.astype(jnp.int32)
    assc = assc_edge_index.astype(jnp.int32)
    id0 = id_t0.astype(jnp.int32)
    id1 = id_t1.astype(jnp.int32)

    wm1 = W_m[:_D]
    wm2 = W_m[_D:]
    wn1 = W_n[:_D]
    wn2 = W_n[_D:]
    wa0 = W_a[:_D]
    wa1 = W_a[_D:]

    s0, d0 = ed0[0], ed0[1]
    s1, d1 = ed1[0], ed1[1]
    sa, da = assc[0], assc[1]

    x0, y0 = _stage_a(reid_t0, scores_t0, wm1)
    x1, y1 = _stage_a(reid_t1, scores_t1, wm1)
    geom = _stage_b(boxes_t0, boxes_t1, s0, d0, s1, d1)
    c0 = _stage_c(geom, 0, W_e, wm2)
    c1 = _stage_c(geom, 1, W_e, wm2)
    aggp = _stage_d(y0, y1, c0, c1, s0, d0, s1, d1)
    p0 = _stage_e(x0, aggp, 0, wn1, wn2, wa0)
    p1 = _stage_e(x1, aggp, 1, wn1, wn2, wa1)
    lg, lab = _stage_f(p0.reshape(_N), p1.reshape(_N), id0, id1, sa, da)
    res = _stage_g(lg, lab)
    return res[0, 0]


# speedup vs baseline: 7.4752x; 1.0009x over previous
import jax
import jax.numpy as jnp
from jax import lax
from jax.experimental import pallas as pl
from jax.experimental.pallas import tpu as pltpu
from jax.experimental.pallas import tpu_sc as plsc

_N = 10000
_E = 160000
_EA = 160000
_D = 128
_DE = 64

_NC = 2
_NS = 16
_NW = _NC * _NS
_CHUNK = 128
_NCHUNK_E = _E // _CHUNK
_NCHUNK_A = _EA // _CHUNK
_ROWBLK = 128
_NB = (_N + _ROWBLK - 1) // _ROWBLK
_NPAD = 10240
_NROWZ = _NPAD // _NS


def _sc_mesh():
    return plsc.VectorSubcoreMesh(core_axis_name="c", subcore_axis_name="s",
                                  num_cores=_NC, num_subcores=_NS)


_SC_PARAMS = pltpu.CompilerParams(needs_layout_passes=False)


def _pack_bf16_pairs(a):
    lo = jax.lax.bitcast_convert_type(
        a[:, :_D // 2].astype(jnp.bfloat16), jnp.uint16).astype(jnp.uint32)
    hi = jax.lax.bitcast_convert_type(
        a[:, _D // 2:].astype(jnp.bfloat16), jnp.uint16).astype(jnp.uint32)
    return jax.lax.bitcast_convert_type(lo | (hi << 16), jnp.int32)


def _xin_y_body(reid_ref, s_ref, wm1_ref, xin_ref, y_ref):
    x = reid_ref[...] * s_ref[...]
    xin_ref[...] = x
    y_ref[...] = jnp.dot(x.astype(jnp.bfloat16), wm1_ref[...],
                         preferred_element_type=jnp.float32)


def _stage_a(reid, scores, wm1):
    return pl.pallas_call(
        _xin_y_body,
        grid=(_NB,),
        in_specs=[pl.BlockSpec((_ROWBLK, _D), lambda i: (i, 0)),
                  pl.BlockSpec((_ROWBLK, 1), lambda i: (i, 0)),
                  pl.BlockSpec((_D, _D), lambda i: (0, 0))],
        out_specs=[pl.BlockSpec((_ROWBLK, _D), lambda i: (i, 0)),
                   pl.BlockSpec((_ROWBLK, _D), lambda i: (i, 0))],
        out_shape=[jax.ShapeDtypeStruct((_N, _D), jnp.float32),
                   jax.ShapeDtypeStruct((_N, _D), jnp.float32)],
    )(reid, scores.reshape(_N, 1), wm1.astype(jnp.bfloat16))


def _geom_body(boxes0, boxes1, s0, d0, s1, d1, geom_hbm,
               table_v, src_v, dst_v, gbuf_v):
    cid = lax.axis_index("c")
    sid = lax.axis_index("s")
    wid = sid * _NC + cid
    for f in range(2):
        bx = boxes0 if f == 0 else boxes1
        sh = s0 if f == 0 else s1
        dh = d0 if f == 0 else d1
        pltpu.sync_copy(bx, table_v)

        def chunk_body(i, _, f=f, sh=sh, dh=dh):
            k = wid + i * _NW
            base = k * _CHUNK
            pltpu.sync_copy(sh.at[pl.ds(base, _CHUNK)], src_v)
            pltpu.sync_copy(dh.at[pl.ds(base, _CHUNK)], dst_v)

            def vec_body(j, _):
                sl = pl.ds(j * 16, 16)
                s16 = src_v[sl] * 4
                d16 = dst_v[sl] * 4
                for comp in range(4):
                    a = plsc.load_gather(table_v, [s16 + comp])
                    b = plsc.load_gather(table_v, [d16 + comp])
                    gbuf_v[comp, sl] = a - b
                return 0

            lax.fori_loop(0, _CHUNK // 16, vec_body, 0)
            pltpu.sync_copy(gbuf_v, geom_hbm.at[f, :, pl.ds(base, _CHUNK)])
            return 0

        nk = (_NCHUNK_E - wid + _NW - 1) // _NW
        lax.fori_loop(0, nk, chunk_body, 0)


def _stage_b(boxes0, boxes1, s0, d0, s1, d1):
    return pl.kernel(
        _geom_body,
        out_type=jax.ShapeDtypeStruct((2, 4, _E), jnp.float32),
        mesh=_sc_mesh(),
        compiler_params=_SC_PARAMS,
        scratch_types=[pltpu.VMEM((_N * 4,), jnp.float32),
                       pltpu.VMEM((_CHUNK,), jnp.int32),
                       pltpu.VMEM((_CHUNK,), jnp.int32),
                       pltpu.VMEM((4, _CHUNK), jnp.float32)],
    )(boxes0.reshape(_N * 4), boxes1.reshape(_N * 4), s0, d0, s1, d1)


_BE = 1280


def _cmat_body(g_ref, we_ref, wm2_ref, c_ref):
    g = g_ref[0].astype(jnp.bfloat16)
    e = lax.dot_general(g, we_ref[...], (((0,), (0,)), ((), ())),
                        preferred_element_type=jnp.float32)
    e = jnp.maximum(e, 0.0).astype(jnp.bfloat16)
    c = jnp.dot(e, wm2_ref[...], preferred_element_type=jnp.float32)
    c_ref[...] = _pack_bf16_pairs(c)


def _stage_c(geom, f, we, wm2):
    return pl.pallas_call(
        _cmat_body,
        grid=(_E // _BE,),
        in_specs=[pl.BlockSpec((1, 4, _BE), lambda i, f=f: (f, 0, i)),
                  pl.BlockSpec((4, _DE), lambda i: (0, 0)),
                  pl.BlockSpec((_DE, _D), lambda i: (0, 0))],
        out_specs=pl.BlockSpec((_BE, _D // 2), lambda i: (i, 0)),
        out_shape=jax.ShapeDtypeStruct((_E, _D // 2), jnp.int32),
    )(geom, we.astype(jnp.bfloat16), wm2.astype(jnp.bfloat16))


_DCH = 64
_DNCH = _E // _DCH
_DPAIRS = (_DNCH // _NW + 2) // 2 + 1


def _agg_body(y0, y1, c0, c1, s0, d0, s1, d1, aggp_hbm,
              acc_sh, mb, sb0, sb1, db0, db1, yb0, yb1, cb0, cb1,
              isem0, isem1, gsem0, gsem1, ssem):
    cid = lax.axis_index("c")
    sid = lax.axis_index("s")
    wid = sid * _NC + cid

    slots = ((sb0, db0, yb0, cb0, isem0, gsem0),
             (sb1, db1, yb1, cb1, isem1, gsem1))

    for f in range(2):
        yh = y0 if f == 0 else y1
        ch = c0 if f == 0 else c1
        sh = s0 if f == 0 else s1
        dh = d0 if f == 0 else d1

        n = (_DNCH - wid + _NW - 1) // _NW

        def base_of(j):
            return (wid + j * _NW) * _DCH

        def fire_idx(j, sl):
            sb, db = sl[0], sl[1]

            @pl.when(j < n)
            def _():
                b = base_of(j)
                pltpu.async_copy(sh.at[pl.ds(b, _DCH)], sb, sl[4])
                pltpu.async_copy(dh.at[pl.ds(b, _DCH)], db, sl[4])

        def wait_idx(j, sl):
            @pl.when(j < n)
            def _():
                pltpu.make_async_copy(sh.at[pl.ds(0, _DCH)], sl[0], sl[4]).wait()
                pltpu.make_async_copy(dh.at[pl.ds(0, _DCH)], sl[1], sl[4]).wait()

        def fire_data(j, sl):
            @pl.when(j < n)
            def _():
                pltpu.async_copy(yh.at[sl[0]], sl[2], sl[5])
                pltpu.async_copy(ch.at[pl.ds(base_of(j), _DCH), :], sl[3], sl[5])

        def wait_data(j, sl):
            @pl.when(j < n)
            def _():
                pltpu.make_async_copy(yh.at[sl[0]], sl[2], sl[5]).wait()
                pltpu.make_async_copy(ch.at[pl.ds(0, _DCH), :], sl[3], sl[5]).wait()

        def compute_scatter(j, sl, sl_prev):
            yb, cb = sl[2], sl[3]

            @pl.when(jnp.logical_and(j >= 1, j < n))
            def _():
                pltpu.make_async_copy(mb, acc_sh.at[sl_prev[1]], ssem).wait()

            @pl.when(j < n)
            def _():
                def row_body(r, _):
                    for g in range(4):
                        gs = pl.ds(g * 16, 16)
                        hs = pl.ds(64 + g * 16, 16)
                        vc = cb[r, gs]
                        lo = yb[r, gs] + plsc.bitcast(vc << 16, jnp.float32)
                        hi = yb[r, hs] + plsc.bitcast(vc, jnp.float32)
                        mb[r, gs] = jnp.maximum(lo, 0.0)
                        mb[r, hs] = jnp.maximum(hi, 0.0)
                    return 0

                lax.fori_loop(0, _DCH, row_body, 0)
                pltpu.async_copy(mb, acc_sh.at[sl[1]], ssem, add=True)

        def zrow(r, _):
            for g in range(8):
                mb[r, pl.ds(g * 16, 16)] = jnp.zeros((16,), jnp.float32)
            return 0

        lax.fori_loop(0, _DCH, zrow, 0)
        for t in range(_NROWZ // _DCH):
            pltpu.sync_copy(
                mb, acc_sh.at[pl.ds(sid * _NROWZ + t * _DCH, _DCH), :])
        plsc.subcore_barrier()

        fire_idx(0, slots[0])
        wait_idx(0, slots[0])
        fire_data(0, slots[0])
        fire_idx(1, slots[1])

        def pair_body(i, _):
            for b in range(2):
                j = 2 * i + b
                wait_idx(j + 1, slots[1 - b])
                fire_data(j + 1, slots[1 - b])
                wait_data(j, slots[b])
                compute_scatter(j, slots[b], slots[1 - b])
                fire_idx(j + 2, slots[b])
            return 0

        lax.fori_loop(0, _DPAIRS, pair_body, 0)

        @pl.when(n > 0)
        def _():
            pltpu.make_async_copy(mb, acc_sh.at[db0], ssem).wait()

        plsc.subcore_barrier()
        pltpu.sync_copy(acc_sh.at[pl.ds(sid * _NROWZ, _NROWZ), :],
                        aggp_hbm.at[f, cid, pl.ds(sid * _NROWZ, _NROWZ), :])
        plsc.subcore_barrier()


def _stage_d(y0, y1, c0, c1, s0, d0, s1, d1):
    return pl.kernel(
        _agg_body,
        out_type=jax.ShapeDtypeStruct((2, _NC, _NPAD, _D), jnp.float32),
        mesh=_sc_mesh(),
        compiler_params=_SC_PARAMS,
        scratch_types=[pltpu.VMEM_SHARED((_NPAD, _D), jnp.float32),
                       pltpu.VMEM((_DCH, _D), jnp.float32),
                       pltpu.VMEM((_DCH,), jnp.int32),
                       pltpu.VMEM((_DCH,), jnp.int32),
                       pltpu.VMEM((_DCH,), jnp.int32),
                       pltpu.VMEM((_DCH,), jnp.int32),
                       pltpu.VMEM((_DCH, _D), jnp.float32),
                       pltpu.VMEM((_DCH, _D), jnp.float32),
                       pltpu.VMEM((_DCH, _D // 2), jnp.int32),
                       pltpu.VMEM((_DCH, _D // 2), jnp.int32),
                       pltpu.SemaphoreType.DMA,
                       pltpu.SemaphoreType.DMA,
                       pltpu.SemaphoreType.DMA,
                       pltpu.SemaphoreType.DMA,
                       pltpu.SemaphoreType.DMA],
    )(y0, y1, c0, c1, s0, d0, s1, d1)


def _h_body(xin_ref, a0_ref, a1_ref, wn1_ref, wn2_ref, wa_ref, p_ref):
    x = xin_ref[...].astype(jnp.bfloat16)
    agg = (a0_ref[0, 0] + a1_ref[0, 0]).astype(jnp.bfloat16)
    pre = (jnp.dot(x, wn1_ref[...], preferred_element_type=jnp.float32)
           + jnp.dot(agg, wn2_ref[...], preferred_element_type=jnp.float32))
    h = jnp.maximum(pre, 0.0)
    nrm = jnp.sqrt(jnp.sum(h * h, axis=1, keepdims=True)) + 1e-8
    hn = h / nrm
    p_ref[...] = jnp.dot(hn.astype(jnp.bfloat16), wa_ref[...],
                         preferred_element_type=jnp.float32)


def _stage_e(xin, aggp, f, wn1, wn2, wa):
    return pl.pallas_call(
        _h_body,
        grid=(_NB,),
        in_specs=[pl.BlockSpec((_ROWBLK, _D), lambda i: (i, 0)),
                  pl.BlockSpec((1, 1, _ROWBLK, _D), lambda i, f=f: (f, 0, i, 0)),
                  pl.BlockSpec((1, 1, _ROWBLK, _D), lambda i, f=f: (f, 1, i, 0)),
                  pl.BlockSpec((_D, _D), lambda i: (0, 0)),
                  pl.BlockSpec((_D, _D), lambda i: (0, 0)),
                  pl.BlockSpec((_D, 1), lambda i: (0, 0))],
        out_specs=pl.BlockSpec((_ROWBLK, 1), lambda i: (i, 0)),
        out_shape=jax.ShapeDtypeStruct((_N, 1), jnp.float32),
    )(xin, aggp, aggp, wn1.astype(jnp.bfloat16), wn2.astype(jnp.bfloat16),
      wa.astype(jnp.bfloat16))


def _assoc_body(p0, p1, id0, id1, sa, da, lg_hbm, lab_hbm,
                p0t, p1t, id0t, id1t, s_v, d_v, lbuf, labbuf):
    cid = lax.axis_index("c")
    sid = lax.axis_index("s")
    wid = sid * _NC + cid
    pltpu.sync_copy(p0, p0t)
    pltpu.sync_copy(p1, p1t)
    pltpu.sync_copy(id0, id0t)
    pltpu.sync_copy(id1, id1t)

    def chunk_body(i, _):
        k = wid + i * _NW
        base = k * _CHUNK
        pltpu.sync_copy(sa.at[pl.ds(base, _CHUNK)], s_v)
        pltpu.sync_copy(da.at[pl.ds(base, _CHUNK)], d_v)

        def vec_body(j, _):
            sl = pl.ds(j * 16, 16)
            s16 = s_v[sl]
            d16 = d_v[sl]
            pa = plsc.load_gather(p0t, [s16])
            pb = plsc.load_gather(p1t, [d16])
            lbuf[sl] = pa + pb
            ia = plsc.load_gather(id0t, [s16])
            ib = plsc.load_gather(id1t, [d16])
            labbuf[sl] = jnp.where(ia == ib, 1.0, 0.0)
            return 0

        lax.fori_loop(0, _CHUNK // 16, vec_body, 0)
        pltpu.sync_copy(lbuf, lg_hbm.at[pl.ds(base, _CHUNK)])
        pltpu.sync_copy(labbuf, lab_hbm.at[pl.ds(base, _CHUNK)])
        return 0

    nk = (_NCHUNK_A - wid + _NW - 1) // _NW
    lax.fori_loop(0, nk, chunk_body, 0)


def _stage_f(p0, p1, id0, id1, sa, da):
    return pl.kernel(
        _assoc_body,
        out_type=(jax.ShapeDtypeStruct((_EA,), jnp.float32),
                  jax.ShapeDtypeStruct((_EA,), jnp.float32)),
        mesh=_sc_mesh(),
        compiler_params=_SC_PARAMS,
        scratch_types=[pltpu.VMEM((_N,), jnp.float32),
                       pltpu.VMEM((_N,), jnp.float32),
                       pltpu.VMEM((_N,), jnp.int32),
                       pltpu.VMEM((_N,), jnp.int32),
                       pltpu.VMEM((_CHUNK,), jnp.int32),
                       pltpu.VMEM((_CHUNK,), jnp.int32),
                       pltpu.VMEM((_CHUNK,), jnp.float32),
                       pltpu.VMEM((_CHUNK,), jnp.float32)],
    )(p0, p1, id0, id1, sa, da)


def _bce_body(lg_ref, lab_ref, out_ref):
    l = lg_ref[...]
    lab = lab_ref[...]
    b = (jnp.maximum(l, 0.0) - l * lab
         + jnp.log(1.0 + jnp.exp(-jnp.abs(l))))
    out_ref[...] = jnp.sum(b, axis=(0, 1), keepdims=True) * (1.0 / _EA)


def _stage_g(lg, lab):
    return pl.pallas_call(
        _bce_body,
        grid=(1,),
        in_specs=[pl.BlockSpec((_NCHUNK_A, _CHUNK), lambda i: (0, 0)),
                  pl.BlockSpec((_NCHUNK_A, _CHUNK), lambda i: (0, 0))],
        out_specs=pl.BlockSpec((1, 1), lambda i: (0, 0)),
        out_shape=jax.ShapeDtypeStruct((1, 1), jnp.float32),
    )(lg.reshape(_NCHUNK_A, _CHUNK), lab.reshape(_NCHUNK_A, _CHUNK))


def kernel(reid_t0, boxes_t0, scores_t0, reid_t1, boxes_t1, scores_t1,
           edge_index_t0, edge_index_t1, assc_edge_index, id_t0, id_t1,
           W_e, W_m, W_n, W_a):
    ed0 = edge_index_t0.astype(jnp.int32)
    ed1 = edge_index_t1.astype(jnp.int32)
    assc = assc_edge_index.astype(jnp.int32)
    id0 = id_t0.astype(jnp.int32)
    id1 = id_t1.astype(jnp.int32)

    wm1 = W_m[:_D]
    wm2 = W_m[_D:]
    wn1 = W_n[:_D]
    wn2 = W_n[_D:]
    wa0 = W_a[:_D]
    wa1 = W_a[_D:]

    s0, d0 = ed0[0], ed0[1]
    s1, d1 = ed1[0], ed1[1]
    sa, da = assc[0], assc[1]

    x0, y0 = _stage_a(reid_t0, scores_t0, wm1)
    x1, y1 = _stage_a(reid_t1, scores_t1, wm1)
    geom = _stage_b(boxes_t0, boxes_t1, s0, d0, s1, d1)
    c0 = _stage_c(geom, 0, W_e, wm2)
    c1 = _stage_c(geom, 1, W_e, wm2)
    aggp = _stage_d(y0, y1, c0, c1, s0, d0, s1, d1)
    p0 = _stage_e(x0, aggp, 0, wn1, wn2, wa0)
    p1 = _stage_e(x1, aggp, 1, wn1, wn2, wa1)
    lg, lab = _stage_f(p0.reshape(_N), p1.reshape(_N), id0, id1, sa, da)
    res = _stage_g(lg, lab)
    return res[0, 0]


# speedup vs baseline: 9.2768x; 1.2410x over previous
import jax
import jax.numpy as jnp
from jax import lax
from jax.experimental import pallas as pl
from jax.experimental.pallas import tpu as pltpu
from jax.experimental.pallas import tpu_sc as plsc

_N = 10000
_E = 160000
_EA = 160000
_D = 128
_DE = 64

_NC = 2
_NS = 16
_NW = _NC * _NS
_CHUNK = 128
_NCHUNK_E = _E // _CHUNK
_NCHUNK_A = _EA // _CHUNK
_ROWBLK = 128
_NB = 79
_NPAD = 10112
_NROWZ = _NPAD // _NS


def _sc_mesh():
    return plsc.VectorSubcoreMesh(core_axis_name="c", subcore_axis_name="s",
                                  num_cores=_NC, num_subcores=_NS)


_SC_PARAMS = pltpu.CompilerParams(needs_layout_passes=False)


def _xin_y_body(reid_ref, s_ref, wm1_ref, xin_ref, y_ref):
    x = reid_ref[...] * s_ref[...]
    xin_ref[...] = x
    y_ref[...] = jnp.dot(x, wm1_ref[...], preferred_element_type=jnp.float32)


def _stage_a(reid, scores, wm1):
    return pl.pallas_call(
        _xin_y_body,
        grid=(_NB,),
        in_specs=[pl.BlockSpec((_ROWBLK, _D), lambda i: (i, 0)),
                  pl.BlockSpec((_ROWBLK, 1), lambda i: (i, 0)),
                  pl.BlockSpec((_D, _D), lambda i: (0, 0))],
        out_specs=[pl.BlockSpec((_ROWBLK, _D), lambda i: (i, 0)),
                   pl.BlockSpec((_ROWBLK, _D), lambda i: (i, 0))],
        out_shape=[jax.ShapeDtypeStruct((_N, _D), jnp.float32),
                   jax.ShapeDtypeStruct((_N, _D), jnp.float32)],
    )(reid, scores.reshape(_N, 1), wm1)


def _geom_body(boxes, sh, dh, geom_hbm, table_v, src_v, dst_v, gbuf_v):
    cid = lax.axis_index("c")
    sid = lax.axis_index("s")
    wid = sid * _NC + cid
    pltpu.sync_copy(boxes, table_v)

    def chunk_body(i, _):
        k = wid + i * _NW
        base = k * _CHUNK
        pltpu.sync_copy(sh.at[pl.ds(base, _CHUNK)], src_v)
        pltpu.sync_copy(dh.at[pl.ds(base, _CHUNK)], dst_v)

        def vec_body(j, _):
            sl = pl.ds(j * 16, 16)
            s16 = src_v[sl] * 4
            d16 = dst_v[sl] * 4
            for comp in range(4):
                a = plsc.load_gather(table_v, [s16 + comp])
                b = plsc.load_gather(table_v, [d16 + comp])
                gbuf_v[comp, sl] = a - b
            return 0

        lax.fori_loop(0, _CHUNK // 16, vec_body, 0)
        pltpu.sync_copy(gbuf_v, geom_hbm.at[:, pl.ds(base, _CHUNK)])
        return 0

    nk = (_NCHUNK_E - wid + _NW - 1) // _NW
    lax.fori_loop(0, nk, chunk_body, 0)


def _stage_b(boxes, s, d):
    return pl.kernel(
        _geom_body,
        out_type=jax.ShapeDtypeStruct((4, _E), jnp.float32),
        mesh=_sc_mesh(),
        compiler_params=_SC_PARAMS,
        scratch_types=[pltpu.VMEM((_N * 4,), jnp.float32),
                       pltpu.VMEM((_CHUNK,), jnp.int32),
                       pltpu.VMEM((_CHUNK,), jnp.int32),
                       pltpu.VMEM((4, _CHUNK), jnp.float32)],
    )(boxes.reshape(_N * 4), s, d)


_BE = 1280


def _pack_rne(c):
    u = jax.lax.bitcast_convert_type(c, jnp.uint32)
    lo = (u[:, :_D // 2] + jnp.uint32(0x8000)) >> 16
    hi = (u[:, _D // 2:] + jnp.uint32(0x8000)) & jnp.uint32(0xFFFF0000)
    return jax.lax.bitcast_convert_type(lo | hi, jnp.int32)


def _cmat_body(g_ref, we_ref, wm2_ref, c_ref):
    g = g_ref[...]
    e = lax.dot_general(g, we_ref[...], (((0,), (0,)), ((), ())),
                        preferred_element_type=jnp.float32)
    e = jnp.maximum(e, 0.0)
    c = jnp.dot(e, wm2_ref[...], preferred_element_type=jnp.float32)
    c_ref[...] = _pack_rne(c)


def _stage_c(geom, we, wm2):
    return pl.pallas_call(
        _cmat_body,
        grid=(_E // _BE,),
        in_specs=[pl.BlockSpec((4, _BE), lambda i: (0, i)),
                  pl.BlockSpec((4, _DE), lambda i: (0, 0)),
                  pl.BlockSpec((_DE, _D), lambda i: (0, 0))],
        out_specs=pl.BlockSpec((_BE, _D // 2), lambda i: (i, 0)),
        out_shape=jax.ShapeDtypeStruct((_E, _D // 2), jnp.int32),
    )(geom, we, wm2)


_DCH = 64
_DNCH = _E // _DCH
_DPAIRS = (_DNCH // _NW + 2) // 2 + 1


def _agg_body(yh, ch, sh, dh, aggp_hbm,
              acc_sh, mb0, mb1, sb0, sb1, db0, db1, yb0, yb1, cb0, cb1,
              isem0, isem1, gsem0, gsem1, ssem0, ssem1):
    cid = lax.axis_index("c")
    sid = lax.axis_index("s")
    wid = sid * _NC + cid

    slots = ((sb0, db0, yb0, cb0, mb0, isem0, gsem0, ssem0),
             (sb1, db1, yb1, cb1, mb1, isem1, gsem1, ssem1))

    n = (_DNCH - wid + _NW - 1) // _NW

    def base_of(j):
        return (wid + j * _NW) * _DCH

    def fire_idx(j, sl):
        @pl.when(j < n)
        def _():
            b = base_of(j)
            pltpu.async_copy(sh.at[pl.ds(b, _DCH)], sl[0], sl[5])
            pltpu.async_copy(dh.at[pl.ds(b, _DCH)], sl[1], sl[5])

    def wait_idx(j, sl):
        @pl.when(j < n)
        def _():
            pltpu.make_async_copy(sh.at[pl.ds(0, _DCH)], sl[0], sl[5]).wait()
            pltpu.make_async_copy(dh.at[pl.ds(0, _DCH)], sl[1], sl[5]).wait()

    def fire_data(j, sl):
        @pl.when(j < n)
        def _():
            pltpu.async_copy(yh.at[sl[0]], sl[2], sl[6])
            pltpu.async_copy(ch.at[pl.ds(base_of(j), _DCH), :], sl[3], sl[6])

    def wait_data(j, sl):
        @pl.when(j < n)
        def _():
            pltpu.make_async_copy(yh.at[sl[0]], sl[2], sl[6]).wait()
            pltpu.make_async_copy(ch.at[pl.ds(0, _DCH), :], sl[3], sl[6]).wait()

    def compute_scatter(j, sl):
        yb, cb, mb = sl[2], sl[3], sl[4]

        @pl.when(jnp.logical_and(j >= 2, j < n))
        def _():
            pltpu.make_async_copy(mb, acc_sh.at[sl[1]], sl[7]).wait()

        @pl.when(j < n)
        def _():
            def row_body(r4, _):
                for rr in range(4):
                    r = r4 * 4 + rr
                    for g in range(4):
                        gs = pl.ds(g * 16, 16)
                        hs = pl.ds(64 + g * 16, 16)
                        vc = cb[r, gs]
                        lo = yb[r, gs] + plsc.bitcast(vc << 16, jnp.float32)
                        hi = yb[r, hs] + plsc.bitcast(vc, jnp.float32)
                        mb[r, gs] = jnp.maximum(lo, 0.0)
                        mb[r, hs] = jnp.maximum(hi, 0.0)
                return 0

            lax.fori_loop(0, _DCH // 4, row_body, 0)
            pltpu.async_copy(mb, acc_sh.at[sl[1]], sl[7], add=True)

    def zrow(r, _):
        for g in range(8):
            mb0[r, pl.ds(g * 16, 16)] = jnp.zeros((16,), jnp.float32)
        return 0

    lax.fori_loop(0, _DCH, zrow, 0)
    for t in range(9):
        pltpu.sync_copy(mb0, acc_sh.at[pl.ds(sid * _NROWZ + t * _DCH, _DCH), :])
    pltpu.sync_copy(mb0.at[pl.ds(0, _NROWZ - 9 * _DCH), :],
                    acc_sh.at[pl.ds(sid * _NROWZ + 9 * _DCH,
                                    _NROWZ - 9 * _DCH), :])
    plsc.subcore_barrier()

    fire_idx(0, slots[0])
    wait_idx(0, slots[0])
    fire_data(0, slots[0])
    fire_idx(1, slots[1])

    def pair_body(i, _):
        for b in range(2):
            j = 2 * i + b
            wait_idx(j + 1, slots[1 - b])
            fire_data(j + 1, slots[1 - b])
            wait_data(j, slots[b])
            compute_scatter(j, slots[b])
            fire_idx(j + 2, slots[b])
        return 0

    lax.fori_loop(0, _DPAIRS, pair_body, 0)

    @pl.when(n > 0)
    def _():
        pltpu.make_async_copy(mb0, acc_sh.at[db0], ssem0).wait()

    @pl.when(n > 1)
    def _():
        pltpu.make_async_copy(mb1, acc_sh.at[db1], ssem1).wait()

    plsc.subcore_barrier()
    pltpu.sync_copy(acc_sh.at[pl.ds(sid * _NROWZ, _NROWZ), :],
                    aggp_hbm.at[cid, pl.ds(sid * _NROWZ, _NROWZ), :])


def _stage_d(y, c, s, d):
    return pl.kernel(
        _agg_body,
        out_type=jax.ShapeDtypeStruct((_NC, _NPAD, _D), jnp.float32),
        mesh=_sc_mesh(),
        compiler_params=_SC_PARAMS,
        scratch_types=[pltpu.VMEM_SHARED((_NPAD, _D), jnp.float32),
                       pltpu.VMEM((_DCH, _D), jnp.float32),
                       pltpu.VMEM((_DCH, _D), jnp.float32),
                       pltpu.VMEM((_DCH,), jnp.int32),
                       pltpu.VMEM((_DCH,), jnp.int32),
                       pltpu.VMEM((_DCH,), jnp.int32),
                       pltpu.VMEM((_DCH,), jnp.int32),
                       pltpu.VMEM((_DCH, _D), jnp.float32),
                       pltpu.VMEM((_DCH, _D), jnp.float32),
                       pltpu.VMEM((_DCH, _D // 2), jnp.int32),
                       pltpu.VMEM((_DCH, _D // 2), jnp.int32),
                       pltpu.SemaphoreType.DMA,
                       pltpu.SemaphoreType.DMA,
                       pltpu.SemaphoreType.DMA,
                       pltpu.SemaphoreType.DMA,
                       pltpu.SemaphoreType.DMA,
                       pltpu.SemaphoreType.DMA],
    )(y, c, s, d)


def _h_body(xin_ref, a0_ref, a1_ref, wn1_ref, wn2_ref, wa_ref, p_ref):
    x = xin_ref[...]
    agg = a0_ref[0] + a1_ref[0]
    pre = (jnp.dot(x, wn1_ref[...], preferred_element_type=jnp.float32)
           + jnp.dot(agg, wn2_ref[...], preferred_element_type=jnp.float32))
    h = jnp.maximum(pre, 0.0)
    nrm = jnp.sqrt(jnp.sum(h * h, axis=1, keepdims=True)) + 1e-8
    hn = h / nrm
    p_ref[...] = jnp.dot(hn, wa_ref[...], preferred_element_type=jnp.float32)


def _stage_e(xin, aggp, wn1, wn2, wa):
    return pl.pallas_call(
        _h_body,
        grid=(_NB,),
        in_specs=[pl.BlockSpec((_ROWBLK, _D), lambda i: (i, 0)),
                  pl.BlockSpec((1, _ROWBLK, _D), lambda i: (0, i, 0)),
                  pl.BlockSpec((1, _ROWBLK, _D), lambda i: (1, i, 0)),
                  pl.BlockSpec((_D, _D), lambda i: (0, 0)),
                  pl.BlockSpec((_D, _D), lambda i: (0, 0)),
                  pl.BlockSpec((_D, 1), lambda i: (0, 0))],
        out_specs=pl.BlockSpec((_ROWBLK, 1), lambda i: (i, 0)),
        out_shape=jax.ShapeDtypeStruct((_N, 1), jnp.float32),
    )(xin, aggp, aggp, wn1, wn2, wa)


def _assoc_body(p0, p1, id0, id1, sa, da, lg_hbm, lab_hbm,
                p0t, p1t, id0t, id1t, s_v, d_v, lbuf, labbuf):
    cid = lax.axis_index("c")
    sid = lax.axis_index("s")
    wid = sid * _NC + cid
    pltpu.sync_copy(p0, p0t)
    pltpu.sync_copy(p1, p1t)
    pltpu.sync_copy(id0, id0t)
    pltpu.sync_copy(id1, id1t)

    def chunk_body(i, _):
        k = wid + i * _NW
        base = k * _CHUNK
        pltpu.sync_copy(sa.at[pl.ds(base, _CHUNK)], s_v)
        pltpu.sync_copy(da.at[pl.ds(base, _CHUNK)], d_v)

        def vec_body(j, _):
            sl = pl.ds(j * 16, 16)
            s16 = s_v[sl]
            d16 = d_v[sl]
            pa = plsc.load_gather(p0t, [s16])
            pb = plsc.load_gather(p1t, [d16])
            lbuf[sl] = pa + pb
            ia = plsc.load_gather(id0t, [s16])
            ib = plsc.load_gather(id1t, [d16])
            labbuf[sl] = jnp.where(ia == ib, 1.0, 0.0)
            return 0

        lax.fori_loop(0, _CHUNK // 16, vec_body, 0)
        pltpu.sync_copy(lbuf, lg_hbm.at[pl.ds(base, _CHUNK)])
        pltpu.sync_copy(labbuf, lab_hbm.at[pl.ds(base, _CHUNK)])
        return 0

    nk = (_NCHUNK_A - wid + _NW - 1) // _NW
    lax.fori_loop(0, nk, chunk_body, 0)


def _stage_f(p0, p1, id0, id1, sa, da):
    return pl.kernel(
        _assoc_body,
        out_type=(jax.ShapeDtypeStruct((_EA,), jnp.float32),
                  jax.ShapeDtypeStruct((_EA,), jnp.float32)),
        mesh=_sc_mesh(),
        compiler_params=_SC_PARAMS,
        scratch_types=[pltpu.VMEM((_N,), jnp.float32),
                       pltpu.VMEM((_N,), jnp.float32),
                       pltpu.VMEM((_N,), jnp.int32),
                       pltpu.VMEM((_N,), jnp.int32),
                       pltpu.VMEM((_CHUNK,), jnp.int32),
                       pltpu.VMEM((_CHUNK,), jnp.int32),
                       pltpu.VMEM((_CHUNK,), jnp.float32),
                       pltpu.VMEM((_CHUNK,), jnp.float32)],
    )(p0, p1, id0, id1, sa, da)


def _bce_body(lg_ref, lab_ref, out_ref):
    l = lg_ref[...]
    lab = lab_ref[...]
    b = (jnp.maximum(l, 0.0) - l * lab
         + jnp.log(1.0 + jnp.exp(-jnp.abs(l))))
    out_ref[...] = jnp.sum(b, axis=(0, 1), keepdims=True) * (1.0 / _EA)


def _stage_g(lg, lab):
    return pl.pallas_call(
        _bce_body,
        grid=(1,),
        in_specs=[pl.BlockSpec((_NCHUNK_A, _CHUNK), lambda i: (0, 0)),
                  pl.BlockSpec((_NCHUNK_A, _CHUNK), lambda i: (0, 0))],
        out_specs=pl.BlockSpec((1, 1), lambda i: (0, 0)),
        out_shape=jax.ShapeDtypeStruct((1, 1), jnp.float32),
    )(lg.reshape(_NCHUNK_A, _CHUNK), lab.reshape(_NCHUNK_A, _CHUNK))


def kernel(reid_t0, boxes_t0, scores_t0, reid_t1, boxes_t1, scores_t1,
           edge_index_t0, edge_index_t1, assc_edge_index, id_t0, id_t1,
           W_e, W_m, W_n, W_a):
    ed0 = edge_index_t0.astype(jnp.int32)
    ed1 = edge_index_t1.astype(jnp.int32)
    assc = assc_edge_index.astype(jnp.int32)
    id0 = id_t0.astype(jnp.int32)
    id1 = id_t1.astype(jnp.int32)

    wm1 = W_m[:_D]
    wm2 = W_m[_D:]
    wn1 = W_n[:_D]
    wn2 = W_n[_D:]
    wa0 = W_a[:_D]
    wa1 = W_a[_D:]

    s0, d0 = ed0[0], ed0[1]
    s1, d1 = ed1[0], ed1[1]
    sa, da = assc[0], assc[1]

    x0, y0 = _stage_a(reid_t0, scores_t0, wm1)
    x1, y1 = _stage_a(reid_t1, scores_t1, wm1)
    geom0 = _stage_b(boxes_t0, s0, d0)
    geom1 = _stage_b(boxes_t1, s1, d1)
    c0 = _stage_c(geom0, W_e, wm2)
    c1 = _stage_c(geom1, W_e, wm2)
    aggp0 = _stage_d(y0, c0, s0, d0)
    aggp1 = _stage_d(y1, c1, s1, d1)
    p0 = _stage_e(x0, aggp0, wn1, wn2, wa0)
    p1 = _stage_e(x1, aggp1, wn1, wn2, wa1)
    lg, lab = _stage_f(p0.reshape(_N), p1.reshape(_N), id0, id1, sa, da)
    res = _stage_g(lg, lab)
    return res[0, 0]


# speedup vs baseline: 9.8321x; 1.0599x over previous
import jax
import jax.numpy as jnp
from jax import lax
from jax.experimental import pallas as pl
from jax.experimental.pallas import tpu as pltpu
from jax.experimental.pallas import tpu_sc as plsc

_N = 10000
_E = 160000
_EA = 160000
_D = 128
_DE = 64

_NC = 2
_NS = 16
_NW = _NC * _NS
_CHUNK = 128
_NCHUNK_E = _E // _CHUNK
_NCHUNK_A = _EA // _CHUNK
_ROWBLK = 128
_NB = 79
_NPAD = 10112
_NROWZ = _NPAD // _NS


def _sc_mesh():
    return plsc.VectorSubcoreMesh(core_axis_name="c", subcore_axis_name="s",
                                  num_cores=_NC, num_subcores=_NS)


_SC_PARAMS = pltpu.CompilerParams(needs_layout_passes=False)


def _xin_y_body(reid_ref, s_ref, wm1_ref, xin_ref, y_ref):
    x = reid_ref[...] * s_ref[...]
    xin_ref[...] = x
    y_ref[...] = jnp.dot(x, wm1_ref[...], preferred_element_type=jnp.float32)


def _stage_a(reid, scores, wm1):
    return pl.pallas_call(
        _xin_y_body,
        grid=(_NB,),
        in_specs=[pl.BlockSpec((_ROWBLK, _D), lambda i: (i, 0)),
                  pl.BlockSpec((_ROWBLK, 1), lambda i: (i, 0)),
                  pl.BlockSpec((_D, _D), lambda i: (0, 0))],
        out_specs=[pl.BlockSpec((_ROWBLK, _D), lambda i: (i, 0)),
                   pl.BlockSpec((_ROWBLK, _D), lambda i: (i, 0))],
        out_shape=[jax.ShapeDtypeStruct((_N, _D), jnp.float32),
                   jax.ShapeDtypeStruct((_N, _D), jnp.float32)],
    )(reid, scores.reshape(_N, 1), wm1)


_BPAIRS = (_NCHUNK_E // _NW + 2) // 2 + 1


def _geom_body(boxes, sh, dh, geom_hbm, table_v,
               s0, s1, d0, d1, g0, g1, isem0, isem1, osem0, osem1):
    cid = lax.axis_index("c")
    sid = lax.axis_index("s")
    wid = sid * _NC + cid
    pltpu.sync_copy(boxes, table_v)

    slots = ((s0, d0, g0, isem0, osem0), (s1, d1, g1, isem1, osem1))
    n = (_NCHUNK_E - wid + _NW - 1) // _NW

    def base_of(j):
        return (wid + j * _NW) * _CHUNK

    def fire_idx(j, sl):
        @pl.when(j < n)
        def _():
            b = base_of(j)
            pltpu.async_copy(sh.at[pl.ds(b, _CHUNK)], sl[0], sl[3])
            pltpu.async_copy(dh.at[pl.ds(b, _CHUNK)], sl[1], sl[3])

    def wait_idx(j, sl):
        @pl.when(j < n)
        def _():
            pltpu.make_async_copy(sh.at[pl.ds(0, _CHUNK)], sl[0], sl[3]).wait()
            pltpu.make_async_copy(dh.at[pl.ds(0, _CHUNK)], sl[1], sl[3]).wait()

    def wait_out(j, sl):
        @pl.when(jnp.logical_and(j >= 2, j - 2 < n))
        def _():
            pltpu.make_async_copy(sl[2], geom_hbm.at[:, pl.ds(0, _CHUNK)],
                                  sl[4]).wait()

    def compute(j, sl):
        @pl.when(j < n)
        def _():
            def vec_body(v, _):
                vsl = pl.ds(v * 16, 16)
                s16 = sl[0][vsl] * 4
                d16 = sl[1][vsl] * 4
                for comp in range(4):
                    a = plsc.load_gather(table_v, [s16 + comp])
                    b = plsc.load_gather(table_v, [d16 + comp])
                    sl[2][comp, vsl] = a - b
                return 0

            lax.fori_loop(0, _CHUNK // 16, vec_body, 0)
            pltpu.async_copy(sl[2], geom_hbm.at[:, pl.ds(base_of(j), _CHUNK)],
                             sl[4])

    fire_idx(0, slots[0])
    fire_idx(1, slots[1])

    def pair_body(i, _):
        for b in range(2):
            j = 2 * i + b
            wait_idx(j, slots[b])
            wait_out(j, slots[b])
            compute(j, slots[b])
            fire_idx(j + 2, slots[b])
        return 0

    lax.fori_loop(0, _BPAIRS, pair_body, 0)


def _stage_b(boxes, s, d):
    return pl.kernel(
        _geom_body,
        out_type=jax.ShapeDtypeStruct((4, _E), jnp.float32),
        mesh=_sc_mesh(),
        compiler_params=_SC_PARAMS,
        scratch_types=[pltpu.VMEM((_N * 4,), jnp.float32),
                       pltpu.VMEM((_CHUNK,), jnp.int32),
                       pltpu.VMEM((_CHUNK,), jnp.int32),
                       pltpu.VMEM((_CHUNK,), jnp.int32),
                       pltpu.VMEM((_CHUNK,), jnp.int32),
                       pltpu.VMEM((4, _CHUNK), jnp.float32),
                       pltpu.VMEM((4, _CHUNK), jnp.float32),
                       pltpu.SemaphoreType.DMA,
                       pltpu.SemaphoreType.DMA,
                       pltpu.SemaphoreType.DMA,
                       pltpu.SemaphoreType.DMA],
    )(boxes.reshape(_N * 4), s, d)


_BE = 1280


def _pack_rne(c):
    u = jax.lax.bitcast_convert_type(c, jnp.uint32)
    lo = (u[:, :_D // 2] + jnp.uint32(0x8000)) >> 16
    hi = (u[:, _D // 2:] + jnp.uint32(0x8000)) & jnp.uint32(0xFFFF0000)
    return jax.lax.bitcast_convert_type(lo | hi, jnp.int32)


def _cmat_body(g_ref, we_ref, wm2_ref, c_ref):
    g = g_ref[...]
    e = lax.dot_general(g, we_ref[...], (((0,), (0,)), ((), ())),
                        preferred_element_type=jnp.float32)
    e = jnp.maximum(e, 0.0)
    c = jnp.dot(e, wm2_ref[...], preferred_element_type=jnp.float32)
    c_ref[...] = _pack_rne(c)


def _stage_c(geom, we, wm2):
    return pl.pallas_call(
        _cmat_body,
        grid=(_E // _BE,),
        in_specs=[pl.BlockSpec((4, _BE), lambda i: (0, i)),
                  pl.BlockSpec((4, _DE), lambda i: (0, 0)),
                  pl.BlockSpec((_DE, _D), lambda i: (0, 0))],
        out_specs=pl.BlockSpec((_BE, _D // 2), lambda i: (i, 0)),
        out_shape=jax.ShapeDtypeStruct((_E, _D // 2), jnp.int32),
    )(geom, we, wm2)


_DCH = 64
_DNCH = _E // _DCH
_DPAIRS = (_DNCH // _NW + 2) // 2 + 1


def _agg_body(yh, ch, sh, dh, aggp_hbm,
              acc_sh, mb0, mb1, sb0, sb1, db0, db1, yb0, yb1, cb0, cb1,
              isem0, isem1, gsem0, gsem1, ssem0, ssem1):
    cid = lax.axis_index("c")
    sid = lax.axis_index("s")
    wid = sid * _NC + cid

    slots = ((sb0, db0, yb0, cb0, mb0, isem0, gsem0, ssem0),
             (sb1, db1, yb1, cb1, mb1, isem1, gsem1, ssem1))

    n = (_DNCH - wid + _NW - 1) // _NW

    def base_of(j):
        return (wid + j * _NW) * _DCH

    def fire_idx(j, sl):
        @pl.when(j < n)
        def _():
            b = base_of(j)
            pltpu.async_copy(sh.at[pl.ds(b, _DCH)], sl[0], sl[5])
            pltpu.async_copy(dh.at[pl.ds(b, _DCH)], sl[1], sl[5])

    def wait_idx(j, sl):
        @pl.when(j < n)
        def _():
            pltpu.make_async_copy(sh.at[pl.ds(0, _DCH)], sl[0], sl[5]).wait()
            pltpu.make_async_copy(dh.at[pl.ds(0, _DCH)], sl[1], sl[5]).wait()

    def fire_data(j, sl):
        @pl.when(j < n)
        def _():
            pltpu.async_copy(yh.at[sl[0]], sl[2], sl[6])
            pltpu.async_copy(ch.at[pl.ds(base_of(j), _DCH), :], sl[3], sl[6])

    def wait_data(j, sl):
        @pl.when(j < n)
        def _():
            pltpu.make_async_copy(yh.at[sl[0]], sl[2], sl[6]).wait()
            pltpu.make_async_copy(ch.at[pl.ds(0, _DCH), :], sl[3], sl[6]).wait()

    def compute_scatter(j, sl):
        yb, cb, mb = sl[2], sl[3], sl[4]

        @pl.when(jnp.logical_and(j >= 2, j < n))
        def _():
            pltpu.make_async_copy(mb, acc_sh.at[sl[1]], sl[7]).wait()

        @pl.when(j < n)
        def _():
            def row_body(r4, _):
                for rr in range(4):
                    r = r4 * 4 + rr
                    for g in range(4):
                        gs = pl.ds(g * 16, 16)
                        hs = pl.ds(64 + g * 16, 16)
                        vc = cb[r, gs]
                        lo = yb[r, gs] + plsc.bitcast(vc << 16, jnp.float32)
                        hi = yb[r, hs] + plsc.bitcast(vc, jnp.float32)
                        mb[r, gs] = jnp.maximum(lo, 0.0)
                        mb[r, hs] = jnp.maximum(hi, 0.0)
                return 0

            lax.fori_loop(0, _DCH // 4, row_body, 0)
            pltpu.async_copy(mb, acc_sh.at[sl[1]], sl[7], add=True)

    def zrow(r, _):
        for g in range(8):
            mb0[r, pl.ds(g * 16, 16)] = jnp.zeros((16,), jnp.float32)
        return 0

    lax.fori_loop(0, _DCH, zrow, 0)
    for t in range(9):
        pltpu.sync_copy(mb0, acc_sh.at[pl.ds(sid * _NROWZ + t * _DCH, _DCH), :])
    pltpu.sync_copy(mb0.at[pl.ds(0, _NROWZ - 9 * _DCH), :],
                    acc_sh.at[pl.ds(sid * _NROWZ + 9 * _DCH,
                                    _NROWZ - 9 * _DCH), :])
    plsc.subcore_barrier()

    fire_idx(0, slots[0])
    wait_idx(0, slots[0])
    fire_data(0, slots[0])
    fire_idx(1, slots[1])

    def pair_body(i, _):
        for b in range(2):
            j = 2 * i + b
            wait_idx(j + 1, slots[1 - b])
            fire_data(j + 1, slots[1 - b])
            wait_data(j, slots[b])
            compute_scatter(j, slots[b])
            fire_idx(j + 2, slots[b])
        return 0

    lax.fori_loop(0, _DPAIRS, pair_body, 0)

    @pl.when(n > 0)
    def _():
        pltpu.make_async_copy(mb0, acc_sh.at[db0], ssem0).wait()

    @pl.when(n > 1)
    def _():
        pltpu.make_async_copy(mb1, acc_sh.at[db1], ssem1).wait()

    plsc.subcore_barrier()
    pltpu.sync_copy(acc_sh.at[pl.ds(sid * _NROWZ, _NROWZ), :],
                    aggp_hbm.at[cid, pl.ds(sid * _NROWZ, _NROWZ), :])


def _stage_d(y, c, s, d):
    return pl.kernel(
        _agg_body,
        out_type=jax.ShapeDtypeStruct((_NC, _NPAD, _D), jnp.float32),
        mesh=_sc_mesh(),
        compiler_params=_SC_PARAMS,
        scratch_types=[pltpu.VMEM_SHARED((_NPAD, _D), jnp.float32),
                       pltpu.VMEM((_DCH, _D), jnp.float32),
                       pltpu.VMEM((_DCH, _D), jnp.float32),
                       pltpu.VMEM((_DCH,), jnp.int32),
                       pltpu.VMEM((_DCH,), jnp.int32),
                       pltpu.VMEM((_DCH,), jnp.int32),
                       pltpu.VMEM((_DCH,), jnp.int32),
                       pltpu.VMEM((_DCH, _D), jnp.float32),
                       pltpu.VMEM((_DCH, _D), jnp.float32),
                       pltpu.VMEM((_DCH, _D // 2), jnp.int32),
                       pltpu.VMEM((_DCH, _D // 2), jnp.int32),
                       pltpu.SemaphoreType.DMA,
                       pltpu.SemaphoreType.DMA,
                       pltpu.SemaphoreType.DMA,
                       pltpu.SemaphoreType.DMA,
                       pltpu.SemaphoreType.DMA,
                       pltpu.SemaphoreType.DMA],
    )(y, c, s, d)


def _h_body(xin_ref, a0_ref, a1_ref, wn1_ref, wn2_ref, wa_ref, p_ref):
    x = xin_ref[...]
    agg = a0_ref[0] + a1_ref[0]
    pre = (jnp.dot(x, wn1_ref[...], preferred_element_type=jnp.float32)
           + jnp.dot(agg, wn2_ref[...], preferred_element_type=jnp.float32))
    h = jnp.maximum(pre, 0.0)
    nrm = jnp.sqrt(jnp.sum(h * h, axis=1, keepdims=True)) + 1e-8
    hn = h / nrm
    p_ref[...] = jnp.dot(hn, wa_ref[...], preferred_element_type=jnp.float32)


def _stage_e(xin, aggp, wn1, wn2, wa):
    return pl.pallas_call(
        _h_body,
        grid=(_NB,),
        in_specs=[pl.BlockSpec((_ROWBLK, _D), lambda i: (i, 0)),
                  pl.BlockSpec((1, _ROWBLK, _D), lambda i: (0, i, 0)),
                  pl.BlockSpec((1, _ROWBLK, _D), lambda i: (1, i, 0)),
                  pl.BlockSpec((_D, _D), lambda i: (0, 0)),
                  pl.BlockSpec((_D, _D), lambda i: (0, 0)),
                  pl.BlockSpec((_D, 1), lambda i: (0, 0))],
        out_specs=pl.BlockSpec((_ROWBLK, 1), lambda i: (i, 0)),
        out_shape=jax.ShapeDtypeStruct((_N, 1), jnp.float32),
    )(xin, aggp, aggp, wn1, wn2, wa)


_FPAIRS = (_NCHUNK_A // _NW + 2) // 2 + 1


def _assoc_body(p0, p1, id0, id1, sa, da, lg_hbm, lab_hbm,
                p0t, p1t, id0t, id1t,
                s0, s1, d0, d1, lg0, lg1, lb0, lb1,
                isem0, isem1, osem0, osem1):
    cid = lax.axis_index("c")
    sid = lax.axis_index("s")
    wid = sid * _NC + cid
    pltpu.sync_copy(p0, p0t)
    pltpu.sync_copy(p1, p1t)
    pltpu.sync_copy(id0, id0t)
    pltpu.sync_copy(id1, id1t)

    slots = ((s0, d0, lg0, lb0, isem0, osem0),
             (s1, d1, lg1, lb1, isem1, osem1))
    n = (_NCHUNK_A - wid + _NW - 1) // _NW

    def base_of(j):
        return (wid + j * _NW) * _CHUNK

    def fire_idx(j, sl):
        @pl.when(j < n)
        def _():
            b = base_of(j)
            pltpu.async_copy(sa.at[pl.ds(b, _CHUNK)], sl[0], sl[4])
            pltpu.async_copy(da.at[pl.ds(b, _CHUNK)], sl[1], sl[4])

    def wait_idx(j, sl):
        @pl.when(j < n)
        def _():
            pltpu.make_async_copy(sa.at[pl.ds(0, _CHUNK)], sl[0], sl[4]).wait()
            pltpu.make_async_copy(da.at[pl.ds(0, _CHUNK)], sl[1], sl[4]).wait()

    def wait_out(j, sl):
        @pl.when(jnp.logical_and(j >= 2, j - 2 < n))
        def _():
            pltpu.make_async_copy(sl[2], lg_hbm.at[pl.ds(0, _CHUNK)],
                                  sl[5]).wait()
            pltpu.make_async_copy(sl[3], lab_hbm.at[pl.ds(0, _CHUNK)],
                                  sl[5]).wait()

    def compute(j, sl):
        @pl.when(j < n)
        def _():
            def vec_body(v, _):
                vsl = pl.ds(v * 16, 16)
                s16 = sl[0][vsl]
                d16 = sl[1][vsl]
                pa = plsc.load_gather(p0t, [s16])
                pb = plsc.load_gather(p1t, [d16])
                sl[2][vsl] = pa + pb
                ia = plsc.load_gather(id0t, [s16])
                ib = plsc.load_gather(id1t, [d16])
                sl[3][vsl] = jnp.where(ia == ib, 1.0, 0.0)
                return 0

            lax.fori_loop(0, _CHUNK // 16, vec_body, 0)
            b = base_of(j)
            pltpu.async_copy(sl[2], lg_hbm.at[pl.ds(b, _CHUNK)], sl[5])
            pltpu.async_copy(sl[3], lab_hbm.at[pl.ds(b, _CHUNK)], sl[5])

    fire_idx(0, slots[0])
    fire_idx(1, slots[1])

    def pair_body(i, _):
        for b in range(2):
            j = 2 * i + b
            wait_idx(j, slots[b])
            wait_out(j, slots[b])
            compute(j, slots[b])
            fire_idx(j + 2, slots[b])
        return 0

    lax.fori_loop(0, _FPAIRS, pair_body, 0)


def _stage_f(p0, p1, id0, id1, sa, da):
    return pl.kernel(
        _assoc_body,
        out_type=(jax.ShapeDtypeStruct((_EA,), jnp.float32),
                  jax.ShapeDtypeStruct((_EA,), jnp.float32)),
        mesh=_sc_mesh(),
        compiler_params=_SC_PARAMS,
        scratch_types=[pltpu.VMEM((_N,), jnp.float32),
                       pltpu.VMEM((_N,), jnp.float32),
                       pltpu.VMEM((_N,), jnp.int32),
                       pltpu.VMEM((_N,), jnp.int32),
                       pltpu.VMEM((_CHUNK,), jnp.int32),
                       pltpu.VMEM((_CHUNK,), jnp.int32),
                       pltpu.VMEM((_CHUNK,), jnp.int32),
                       pltpu.VMEM((_CHUNK,), jnp.int32),
                       pltpu.VMEM((_CHUNK,), jnp.float32),
                       pltpu.VMEM((_CHUNK,), jnp.float32),
                       pltpu.VMEM((_CHUNK,), jnp.float32),
                       pltpu.VMEM((_CHUNK,), jnp.float32),
                       pltpu.SemaphoreType.DMA,
                       pltpu.SemaphoreType.DMA,
                       pltpu.SemaphoreType.DMA,
                       pltpu.SemaphoreType.DMA],
    )(p0, p1, id0, id1, sa, da)


def _bce_body(lg_ref, lab_ref, out_ref):
    l = lg_ref[...]
    lab = lab_ref[...]
    b = (jnp.maximum(l, 0.0) - l * lab
         + jnp.log(1.0 + jnp.exp(-jnp.abs(l))))
    out_ref[...] = jnp.sum(b, axis=(0, 1), keepdims=True) * (1.0 / _EA)


def _stage_g(lg, lab):
    return pl.pallas_call(
        _bce_body,
        grid=(1,),
        in_specs=[pl.BlockSpec((_NCHUNK_A, _CHUNK), lambda i: (0, 0)),
                  pl.BlockSpec((_NCHUNK_A, _CHUNK), lambda i: (0, 0))],
        out_specs=pl.BlockSpec((1, 1), lambda i: (0, 0)),
        out_shape=jax.ShapeDtypeStruct((1, 1), jnp.float32),
    )(lg.reshape(_NCHUNK_A, _CHUNK), lab.reshape(_NCHUNK_A, _CHUNK))


def kernel(reid_t0, boxes_t0, scores_t0, reid_t1, boxes_t1, scores_t1,
           edge_index_t0, edge_index_t1, assc_edge_index, id_t0, id_t1,
           W_e, W_m, W_n, W_a):
    ed0 = edge_index_t0.astype(jnp.int32)
    ed1 = edge_index_t1.astype(jnp.int32)
    assc = assc_edge_index.astype(jnp.int32)
    id0 = id_t0.astype(jnp.int32)
    id1 = id_t1.astype(jnp.int32)

    wm1 = W_m[:_D]
    wm2 = W_m[_D:]
    wn1 = W_n[:_D]
    wn2 = W_n[_D:]
    wa0 = W_a[:_D]
    wa1 = W_a[_D:]

    s0, d0 = ed0[0], ed0[1]
    s1, d1 = ed1[0], ed1[1]
    sa, da = assc[0], assc[1]

    x0, y0 = _stage_a(reid_t0, scores_t0, wm1)
    x1, y1 = _stage_a(reid_t1, scores_t1, wm1)
    geom0 = _stage_b(boxes_t0, s0, d0)
    geom1 = _stage_b(boxes_t1, s1, d1)
    c0 = _stage_c(geom0, W_e, wm2)
    c1 = _stage_c(geom1, W_e, wm2)
    aggp0 = _stage_d(y0, c0, s0, d0)
    aggp1 = _stage_d(y1, c1, s1, d1)
    p0 = _stage_e(x0, aggp0, wn1, wn2, wa0)
    p1 = _stage_e(x1, aggp1, wn1, wn2, wa1)
    lg, lab = _stage_f(p0.reshape(_N), p1.reshape(_N), id0, id1, sa, da)
    res = _stage_g(lg, lab)
    return res[0, 0]


# speedup vs baseline: 9.9682x; 1.0138x over previous
import jax
import jax.numpy as jnp
from jax import lax
from jax.experimental import pallas as pl
from jax.experimental.pallas import tpu as pltpu
from jax.experimental.pallas import tpu_sc as plsc

_N = 10000
_E = 160000
_EA = 160000
_D = 128
_DE = 64

_NC = 2
_NS = 16
_NW = _NC * _NS
_CHUNK = 128
_NCHUNK_E = _E // _CHUNK
_NCHUNK_A = _EA // _CHUNK
_ROWBLK = 128
_NB = 79
_NPAD = 10112
_NROWZ = _NPAD // _NS


def _sc_mesh():
    return plsc.VectorSubcoreMesh(core_axis_name="c", subcore_axis_name="s",
                                  num_cores=_NC, num_subcores=_NS)


_SC_PARAMS = pltpu.CompilerParams(needs_layout_passes=False)


def _xin_y_body(reid_ref, s_ref, wm1_ref, xin_ref, y_ref):
    x = reid_ref[...] * s_ref[...]
    xin_ref[...] = x
    y_ref[...] = jnp.dot(x, wm1_ref[...], preferred_element_type=jnp.float32)


def _stage_a(reid, scores, wm1):
    return pl.pallas_call(
        _xin_y_body,
        grid=(_NB,),
        in_specs=[pl.BlockSpec((_ROWBLK, _D), lambda i: (i, 0)),
                  pl.BlockSpec((_ROWBLK, 1), lambda i: (i, 0)),
                  pl.BlockSpec((_D, _D), lambda i: (0, 0))],
        out_specs=[pl.BlockSpec((_ROWBLK, _D), lambda i: (i, 0)),
                   pl.BlockSpec((_ROWBLK, _D), lambda i: (i, 0))],
        out_shape=[jax.ShapeDtypeStruct((_N, _D), jnp.float32),
                   jax.ShapeDtypeStruct((_N, _D), jnp.float32)],
    )(reid, scores.reshape(_N, 1), wm1)


_BPAIRS = (_NCHUNK_E // _NW + 2) // 2 + 1


def _geom_body(boxes0, boxes1, sh0, dh0, sh1, dh1, geom_hbm, table_v,
               s0, s1, d0, d1, g0, g1, isem0, isem1, osem0, osem1):
    cid = lax.axis_index("c")
    sid = lax.axis_index("s")
    wid = sid * _NC + cid

    slots = ((s0, d0, g0, isem0, osem0), (s1, d1, g1, isem1, osem1))
    n = (_NCHUNK_E - wid + _NW - 1) // _NW

    def base_of(j):
        return (wid + j * _NW) * _CHUNK

    for f in range(2):
        sh = sh0 if f == 0 else sh1
        dh = dh0 if f == 0 else dh1
        pltpu.sync_copy(boxes0 if f == 0 else boxes1, table_v)

        def fire_idx(j, sl, sh=sh, dh=dh):
            @pl.when(j < n)
            def _():
                b = base_of(j)
                pltpu.async_copy(sh.at[pl.ds(b, _CHUNK)], sl[0], sl[3])
                pltpu.async_copy(dh.at[pl.ds(b, _CHUNK)], sl[1], sl[3])

        def wait_idx(j, sl, sh=sh, dh=dh):
            @pl.when(j < n)
            def _():
                pltpu.make_async_copy(sh.at[pl.ds(0, _CHUNK)], sl[0],
                                      sl[3]).wait()
                pltpu.make_async_copy(dh.at[pl.ds(0, _CHUNK)], sl[1],
                                      sl[3]).wait()

        def wait_out(j, sl, f=f):
            @pl.when(jnp.logical_and(j >= 2, j - 2 < n))
            def _():
                pltpu.make_async_copy(sl[2],
                                      geom_hbm.at[f, :, pl.ds(0, _CHUNK)],
                                      sl[4]).wait()

        def compute(j, sl, f=f):
            @pl.when(j < n)
            def _():
                def vec_body(v, _):
                    vsl = pl.ds(v * 16, 16)
                    s16 = sl[0][vsl] * 4
                    d16 = sl[1][vsl] * 4
                    for comp in range(4):
                        a = plsc.load_gather(table_v, [s16 + comp])
                        b = plsc.load_gather(table_v, [d16 + comp])
                        sl[2][comp, vsl] = a - b
                    return 0

                lax.fori_loop(0, _CHUNK // 16, vec_body, 0)
                pltpu.async_copy(sl[2],
                                 geom_hbm.at[f, :, pl.ds(base_of(j), _CHUNK)],
                                 sl[4])

        fire_idx(0, slots[0])
        fire_idx(1, slots[1])

        def pair_body(i, _):
            for b in range(2):
                j = 2 * i + b
                wait_idx(j, slots[b])
                wait_out(j, slots[b])
                compute(j, slots[b])
                fire_idx(j + 2, slots[b])
            return 0

        lax.fori_loop(0, _BPAIRS, pair_body, 0)


def _stage_b(boxes0, boxes1, s0h, d0h, s1h, d1h):
    return pl.kernel(
        _geom_body,
        out_type=jax.ShapeDtypeStruct((2, 4, _E), jnp.float32),
        mesh=_sc_mesh(),
        compiler_params=_SC_PARAMS,
        scratch_types=[pltpu.VMEM((_N * 4,), jnp.float32),
                       pltpu.VMEM((_CHUNK,), jnp.int32),
                       pltpu.VMEM((_CHUNK,), jnp.int32),
                       pltpu.VMEM((_CHUNK,), jnp.int32),
                       pltpu.VMEM((_CHUNK,), jnp.int32),
                       pltpu.VMEM((4, _CHUNK), jnp.float32),
                       pltpu.VMEM((4, _CHUNK), jnp.float32),
                       pltpu.SemaphoreType.DMA,
                       pltpu.SemaphoreType.DMA,
                       pltpu.SemaphoreType.DMA,
                       pltpu.SemaphoreType.DMA],
    )(boxes0.reshape(_N * 4), boxes1.reshape(_N * 4), s0h, d0h, s1h, d1h)


_BE = 1280


def _pack_rne(c):
    u = jax.lax.bitcast_convert_type(c, jnp.uint32)
    lo = (u[:, :_D // 2] + jnp.uint32(0x8000)) >> 16
    hi = (u[:, _D // 2:] + jnp.uint32(0x8000)) & jnp.uint32(0xFFFF0000)
    return jax.lax.bitcast_convert_type(lo | hi, jnp.int32)


def _cmat_body(g_ref, we_ref, wm2_ref, c_ref):
    g = g_ref[0]
    e = lax.dot_general(g, we_ref[...], (((0,), (0,)), ((), ())),
                        preferred_element_type=jnp.float32)
    e = jnp.maximum(e, 0.0)
    c = jnp.dot(e, wm2_ref[...], preferred_element_type=jnp.float32)
    c_ref[...] = _pack_rne(c)


def _stage_c(geom, f, we, wm2):
    return pl.pallas_call(
        _cmat_body,
        grid=(_E // _BE,),
        in_specs=[pl.BlockSpec((1, 4, _BE), lambda i, f=f: (f, 0, i)),
                  pl.BlockSpec((4, _DE), lambda i: (0, 0)),
                  pl.BlockSpec((_DE, _D), lambda i: (0, 0))],
        out_specs=pl.BlockSpec((_BE, _D // 2), lambda i: (i, 0)),
        out_shape=jax.ShapeDtypeStruct((_E, _D // 2), jnp.int32),
    )(geom, we, wm2)


_DCH = 64
_DNCH = _E // _DCH
_DPAIRS = (_DNCH // _NW + 2) // 2 + 1


def _agg_body(yh, ch, sh, dh, aggp_hbm,
              acc_sh, mb0, mb1, sb0, sb1, db0, db1, yb0, yb1, cb0, cb1,
              isem0, isem1, gsem0, gsem1, ssem0, ssem1):
    cid = lax.axis_index("c")
    sid = lax.axis_index("s")
    wid = sid * _NC + cid

    slots = ((sb0, db0, yb0, cb0, mb0, isem0, gsem0, ssem0),
             (sb1, db1, yb1, cb1, mb1, isem1, gsem1, ssem1))

    n = (_DNCH - wid + _NW - 1) // _NW

    def base_of(j):
        return (wid + j * _NW) * _DCH

    def fire_idx(j, sl):
        @pl.when(j < n)
        def _():
            b = base_of(j)
            pltpu.async_copy(sh.at[pl.ds(b, _DCH)], sl[0], sl[5])
            pltpu.async_copy(dh.at[pl.ds(b, _DCH)], sl[1], sl[5])

    def wait_idx(j, sl):
        @pl.when(j < n)
        def _():
            pltpu.make_async_copy(sh.at[pl.ds(0, _DCH)], sl[0], sl[5]).wait()
            pltpu.make_async_copy(dh.at[pl.ds(0, _DCH)], sl[1], sl[5]).wait()

    def fire_data(j, sl):
        @pl.when(j < n)
        def _():
            pltpu.async_copy(yh.at[sl[0]], sl[2], sl[6])
            pltpu.async_copy(ch.at[pl.ds(base_of(j), _DCH), :], sl[3], sl[6])

    def wait_data(j, sl):
        @pl.when(j < n)
        def _():
            pltpu.make_async_copy(yh.at[sl[0]], sl[2], sl[6]).wait()
            pltpu.make_async_copy(ch.at[pl.ds(0, _DCH), :], sl[3], sl[6]).wait()

    def compute_scatter(j, sl):
        yb, cb, mb = sl[2], sl[3], sl[4]

        @pl.when(jnp.logical_and(j >= 2, j < n))
        def _():
            pltpu.make_async_copy(mb, acc_sh.at[sl[1]], sl[7]).wait()

        @pl.when(j < n)
        def _():
            def row_body(r4, _):
                for rr in range(4):
                    r = r4 * 4 + rr
                    for g in range(4):
                        gs = pl.ds(g * 16, 16)
                        hs = pl.ds(64 + g * 16, 16)
                        vc = cb[r, gs]
                        lo = yb[r, gs] + plsc.bitcast(vc << 16, jnp.float32)
                        hi = yb[r, hs] + plsc.bitcast(vc, jnp.float32)
                        mb[r, gs] = jnp.maximum(lo, 0.0)
                        mb[r, hs] = jnp.maximum(hi, 0.0)
                return 0

            lax.fori_loop(0, _DCH // 4, row_body, 0)
            pltpu.async_copy(mb, acc_sh.at[sl[1]], sl[7], add=True)

    def zrow(r, _):
        for g in range(8):
            mb0[r, pl.ds(g * 16, 16)] = jnp.zeros((16,), jnp.float32)
        return 0

    lax.fori_loop(0, _DCH, zrow, 0)
    for t in range(9):
        pltpu.sync_copy(mb0, acc_sh.at[pl.ds(sid * _NROWZ + t * _DCH, _DCH), :])
    pltpu.sync_copy(mb0.at[pl.ds(0, _NROWZ - 9 * _DCH), :],
                    acc_sh.at[pl.ds(sid * _NROWZ + 9 * _DCH,
                                    _NROWZ - 9 * _DCH), :])
    plsc.subcore_barrier()

    fire_idx(0, slots[0])
    wait_idx(0, slots[0])
    fire_data(0, slots[0])
    fire_idx(1, slots[1])

    def pair_body(i, _):
        for b in range(2):
            j = 2 * i + b
            wait_idx(j + 1, slots[1 - b])
            fire_data(j + 1, slots[1 - b])
            wait_data(j, slots[b])
            compute_scatter(j, slots[b])
            fire_idx(j + 2, slots[b])
        return 0

    lax.fori_loop(0, _DPAIRS, pair_body, 0)

    @pl.when(n > 0)
    def _():
        pltpu.make_async_copy(mb0, acc_sh.at[db0], ssem0).wait()

    @pl.when(n > 1)
    def _():
        pltpu.make_async_copy(mb1, acc_sh.at[db1], ssem1).wait()

    plsc.subcore_barrier()
    pltpu.sync_copy(acc_sh.at[pl.ds(sid * _NROWZ, _NROWZ), :],
                    aggp_hbm.at[cid, pl.ds(sid * _NROWZ, _NROWZ), :])


def _stage_d(y, c, s, d):
    return pl.kernel(
        _agg_body,
        out_type=jax.ShapeDtypeStruct((_NC, _NPAD, _D), jnp.float32),
        mesh=_sc_mesh(),
        compiler_params=_SC_PARAMS,
        scratch_types=[pltpu.VMEM_SHARED((_NPAD, _D), jnp.float32),
                       pltpu.VMEM((_DCH, _D), jnp.float32),
                       pltpu.VMEM((_DCH, _D), jnp.float32),
                       pltpu.VMEM((_DCH,), jnp.int32),
                       pltpu.VMEM((_DCH,), jnp.int32),
                       pltpu.VMEM((_DCH,), jnp.int32),
                       pltpu.VMEM((_DCH,), jnp.int32),
                       pltpu.VMEM((_DCH, _D), jnp.float32),
                       pltpu.VMEM((_DCH, _D), jnp.float32),
                       pltpu.VMEM((_DCH, _D // 2), jnp.int32),
                       pltpu.VMEM((_DCH, _D // 2), jnp.int32),
                       pltpu.SemaphoreType.DMA,
                       pltpu.SemaphoreType.DMA,
                       pltpu.SemaphoreType.DMA,
                       pltpu.SemaphoreType.DMA,
                       pltpu.SemaphoreType.DMA,
                       pltpu.SemaphoreType.DMA],
    )(y, c, s, d)


def _h_body(xin_ref, a0_ref, a1_ref, wn1_ref, wn2_ref, wa_ref, p_ref):
    x = xin_ref[...]
    agg = a0_ref[0] + a1_ref[0]
    pre = (jnp.dot(x, wn1_ref[...], preferred_element_type=jnp.float32)
           + jnp.dot(agg, wn2_ref[...], preferred_element_type=jnp.float32))
    h = jnp.maximum(pre, 0.0)
    nrm = jnp.sqrt(jnp.sum(h * h, axis=1, keepdims=True)) + 1e-8
    hn = h / nrm
    p_ref[...] = jnp.dot(hn, wa_ref[...], preferred_element_type=jnp.float32)


def _stage_e(xin, aggp, wn1, wn2, wa):
    return pl.pallas_call(
        _h_body,
        grid=(_NB,),
        in_specs=[pl.BlockSpec((_ROWBLK, _D), lambda i: (i, 0)),
                  pl.BlockSpec((1, _ROWBLK, _D), lambda i: (0, i, 0)),
                  pl.BlockSpec((1, _ROWBLK, _D), lambda i: (1, i, 0)),
                  pl.BlockSpec((_D, _D), lambda i: (0, 0)),
                  pl.BlockSpec((_D, _D), lambda i: (0, 0)),
                  pl.BlockSpec((_D, 1), lambda i: (0, 0))],
        out_specs=pl.BlockSpec((_ROWBLK, 1), lambda i: (i, 0)),
        out_shape=jax.ShapeDtypeStruct((_N, 1), jnp.float32),
    )(xin, aggp, aggp, wn1, wn2, wa)


_FPAIRS = (_NCHUNK_A // _NW + 2) // 2 + 1


def _assoc_body(p0, p1, id0, id1, sa, da, lg_hbm, lab_hbm,
                p0t, p1t, id0t, id1t,
                s0, s1, d0, d1, lg0, lg1, lb0, lb1,
                isem0, isem1, osem0, osem1):
    cid = lax.axis_index("c")
    sid = lax.axis_index("s")
    wid = sid * _NC + cid
    pltpu.sync_copy(p0, p0t)
    pltpu.sync_copy(p1, p1t)
    pltpu.sync_copy(id0, id0t)
    pltpu.sync_copy(id1, id1t)

    slots = ((s0, d0, lg0, lb0, isem0, osem0),
             (s1, d1, lg1, lb1, isem1, osem1))
    n = (_NCHUNK_A - wid + _NW - 1) // _NW

    def base_of(j):
        return (wid + j * _NW) * _CHUNK

    def fire_idx(j, sl):
        @pl.when(j < n)
        def _():
            b = base_of(j)
            pltpu.async_copy(sa.at[pl.ds(b, _CHUNK)], sl[0], sl[4])
            pltpu.async_copy(da.at[pl.ds(b, _CHUNK)], sl[1], sl[4])

    def wait_idx(j, sl):
        @pl.when(j < n)
        def _():
            pltpu.make_async_copy(sa.at[pl.ds(0, _CHUNK)], sl[0], sl[4]).wait()
            pltpu.make_async_copy(da.at[pl.ds(0, _CHUNK)], sl[1], sl[4]).wait()

    def wait_out(j, sl):
        @pl.when(jnp.logical_and(j >= 2, j - 2 < n))
        def _():
            pltpu.make_async_copy(sl[2], lg_hbm.at[pl.ds(0, _CHUNK)],
                                  sl[5]).wait()
            pltpu.make_async_copy(sl[3], lab_hbm.at[pl.ds(0, _CHUNK)],
                                  sl[5]).wait()

    def compute(j, sl):
        @pl.when(j < n)
        def _():
            def vec_body(v, _):
                vsl = pl.ds(v * 16, 16)
                s16 = sl[0][vsl]
                d16 = sl[1][vsl]
                pa = plsc.load_gather(p0t, [s16])
                pb = plsc.load_gather(p1t, [d16])
                sl[2][vsl] = pa + pb
                ia = plsc.load_gather(id0t, [s16])
                ib = plsc.load_gather(id1t, [d16])
                sl[3][vsl] = jnp.where(ia == ib, 1.0, 0.0)
                return 0

            lax.fori_loop(0, _CHUNK // 16, vec_body, 0)
            b = base_of(j)
            pltpu.async_copy(sl[2], lg_hbm.at[pl.ds(b, _CHUNK)], sl[5])
            pltpu.async_copy(sl[3], lab_hbm.at[pl.ds(b, _CHUNK)], sl[5])

    fire_idx(0, slots[0])
    fire_idx(1, slots[1])

    def pair_body(i, _):
        for b in range(2):
            j = 2 * i + b
            wait_idx(j, slots[b])
            wait_out(j, slots[b])
            compute(j, slots[b])
            fire_idx(j + 2, slots[b])
        return 0

    lax.fori_loop(0, _FPAIRS, pair_body, 0)


def _stage_f(p0, p1, id0, id1, sa, da):
    return pl.kernel(
        _assoc_body,
        out_type=(jax.ShapeDtypeStruct((_EA,), jnp.float32),
                  jax.ShapeDtypeStruct((_EA,), jnp.float32)),
        mesh=_sc_mesh(),
        compiler_params=_SC_PARAMS,
        scratch_types=[pltpu.VMEM((_N,), jnp.float32),
                       pltpu.VMEM((_N,), jnp.float32),
                       pltpu.VMEM((_N,), jnp.int32),
                       pltpu.VMEM((_N,), jnp.int32),
                       pltpu.VMEM((_CHUNK,), jnp.int32),
                       pltpu.VMEM((_CHUNK,), jnp.int32),
                       pltpu.VMEM((_CHUNK,), jnp.int32),
                       pltpu.VMEM((_CHUNK,), jnp.int32),
                       pltpu.VMEM((_CHUNK,), jnp.float32),
                       pltpu.VMEM((_CHUNK,), jnp.float32),
                       pltpu.VMEM((_CHUNK,), jnp.float32),
                       pltpu.VMEM((_CHUNK,), jnp.float32),
                       pltpu.SemaphoreType.DMA,
                       pltpu.SemaphoreType.DMA,
                       pltpu.SemaphoreType.DMA,
                       pltpu.SemaphoreType.DMA],
    )(p0, p1, id0, id1, sa, da)


def _bce_body(lg_ref, lab_ref, out_ref):
    l = lg_ref[...]
    lab = lab_ref[...]
    b = (jnp.maximum(l, 0.0) - l * lab
         + jnp.log(1.0 + jnp.exp(-jnp.abs(l))))
    out_ref[...] = jnp.sum(b, axis=(0, 1), keepdims=True) * (1.0 / _EA)


def _stage_g(lg, lab):
    return pl.pallas_call(
        _bce_body,
        grid=(1,),
        in_specs=[pl.BlockSpec((_NCHUNK_A, _CHUNK), lambda i: (0, 0)),
                  pl.BlockSpec((_NCHUNK_A, _CHUNK), lambda i: (0, 0))],
        out_specs=pl.BlockSpec((1, 1), lambda i: (0, 0)),
        out_shape=jax.ShapeDtypeStruct((1, 1), jnp.float32),
    )(lg.reshape(_NCHUNK_A, _CHUNK), lab.reshape(_NCHUNK_A, _CHUNK))


def kernel(reid_t0, boxes_t0, scores_t0, reid_t1, boxes_t1, scores_t1,
           edge_index_t0, edge_index_t1, assc_edge_index, id_t0, id_t1,
           W_e, W_m, W_n, W_a):
    ed0 = edge_index_t0.astype(jnp.int32)
    ed1 = edge_index_t1.astype(jnp.int32)
    assc = assc_edge_index.astype(jnp.int32)
    id0 = id_t0.astype(jnp.int32)
    id1 = id_t1.astype(jnp.int32)

    wm1 = W_m[:_D]
    wm2 = W_m[_D:]
    wn1 = W_n[:_D]
    wn2 = W_n[_D:]
    wa0 = W_a[:_D]
    wa1 = W_a[_D:]

    s0, d0 = ed0[0], ed0[1]
    s1, d1 = ed1[0], ed1[1]
    sa, da = assc[0], assc[1]

    x0, y0 = _stage_a(reid_t0, scores_t0, wm1)
    x1, y1 = _stage_a(reid_t1, scores_t1, wm1)
    geom = _stage_b(boxes_t0, boxes_t1, s0, d0, s1, d1)
    c0 = _stage_c(geom, 0, W_e, wm2)
    c1 = _stage_c(geom, 1, W_e, wm2)
    aggp0 = _stage_d(y0, c0, s0, d0)
    aggp1 = _stage_d(y1, c1, s1, d1)
    p0 = _stage_e(x0, aggp0, wn1, wn2, wa0)
    p1 = _stage_e(x1, aggp1, wn1, wn2, wa1)
    lg, lab = _stage_f(p0.reshape(_N), p1.reshape(_N), id0, id1, sa, da)
    res = _stage_g(lg, lab)
    return res[0, 0]
